# Initial kernel scaffold; baseline (speedup 1.0000x reference)
import functools

import jax
import jax.numpy as jnp
from jax import lax
from jax.experimental import pallas as pl
from jax.experimental.pallas import tpu as pltpu
from jax.experimental.pallas import tpu_sc as plsc

NF = 64
HALF = 32
NC = 2
NS = 16
BN = 2000
BE = 2000
CB = 5000
CD = 2000


def _full_spec(shape):
    return pl.BlockSpec(shape, lambda i: tuple(0 for _ in shape))


def _node_encode_body(nt_ref, w0_ref, b0_ref, w1_ref, b1_ref, wpa_ref,
                      bpa_ref, out_ref):
    h = jnp.maximum(jnp.dot(nt_ref[...], w0_ref[...],
                            preferred_element_type=jnp.float32) + b0_ref[...], 0.0)
    pe = jnp.maximum(jnp.dot(h, w1_ref[...],
                             preferred_element_type=jnp.float32) + b1_ref[...], 0.0)
    out_ref[...] = jnp.dot(pe, wpa_ref[...],
                           preferred_element_type=jnp.float32) + bpa_ref[...]


def _rel_encode_body(g_ref, ra_ref, w0_ref, wra_ref, b0_ref, w1_ref, b1_ref,
                     w2_ref, b2_ref, walo_ref, wahi_ref, blo_ref, bhi_ref,
                     out_ref):
    x = (jnp.dot(g_ref[...], w0_ref[...], preferred_element_type=jnp.float32)
         + ra_ref[...] * wra_ref[...] + b0_ref[...])
    h = jnp.maximum(x, 0.0)
    h = jnp.maximum(jnp.dot(h, w1_ref[...],
                            preferred_element_type=jnp.float32) + b1_ref[...], 0.0)
    rel = jnp.maximum(jnp.dot(h, w2_ref[...],
                              preferred_element_type=jnp.float32) + b2_ref[...], 0.0)
    out_ref[0] = jnp.dot(rel, walo_ref[...],
                         preferred_element_type=jnp.float32) + blo_ref[...]
    out_ref[1] = jnp.dot(rel, wahi_ref[...],
                         preferred_element_type=jnp.float32) + bhi_ref[...]


def _prop_node_body(agg_ref, pec_ref, pblo_ref, pbhi_ref, w1lo_ref, w1hi_ref,
                    w2lo_ref, w2hi_ref, p1_ref, p2_ref):
    x = (pec_ref[...]
         + jnp.dot(agg_ref[0], pblo_ref[...], preferred_element_type=jnp.float32)
         + jnp.dot(agg_ref[1], pbhi_ref[...], preferred_element_type=jnp.float32))
    pe = jnp.maximum(x, 0.0)
    p1_ref[0] = jnp.dot(pe, w1lo_ref[...], preferred_element_type=jnp.float32)
    p1_ref[1] = jnp.dot(pe, w1hi_ref[...], preferred_element_type=jnp.float32)
    p2_ref[0] = jnp.dot(pe, w2lo_ref[...], preferred_element_type=jnp.float32)
    p2_ref[1] = jnp.dot(pe, w2hi_ref[...], preferred_element_type=jnp.float32)


def _final_body(agg_ref, pec_ref, pblo_ref, pbhi_ref, fw0_ref, fb0_ref,
                fw1_ref, fb1_ref, fw2_ref, fb2_ref, out_ref):
    x = (pec_ref[...]
         + jnp.dot(agg_ref[0], pblo_ref[...], preferred_element_type=jnp.float32)
         + jnp.dot(agg_ref[1], pbhi_ref[...], preferred_element_type=jnp.float32))
    pe = jnp.maximum(x, 0.0)
    p = jnp.maximum(jnp.dot(pe, fw0_ref[...],
                            preferred_element_type=jnp.float32) + fb0_ref[...], 0.0)
    p = jnp.maximum(jnp.dot(p, fw1_ref[...],
                            preferred_element_type=jnp.float32) + fb1_ref[...], 0.0)
    out_ref[...] = jnp.dot(p, fw2_ref[...],
                           preferred_element_type=jnp.float32) + fb2_ref[...]


def _relu_chunk(buf, rows):
    def body(i, carry):
        a = buf[i, pl.ds(0, 16)]
        buf[i, pl.ds(0, 16)] = jnp.maximum(a, 0.0)
        b = buf[i, pl.ds(16, 16)]
        buf[i, pl.ds(16, 16)] = jnp.maximum(b, 0.0)
        return carry
    lax.fori_loop(0, rows, body, 0)


def _make_gather_kernel(n, e):
    epw = e // (NC * NS)
    nchunks = epw // CB
    mesh = plsc.VectorSubcoreMesh(core_axis_name="c", subcore_axis_name="s")

    @functools.partial(
        pl.kernel,
        out_type=jax.ShapeDtypeStruct((e, 2 * 16), jnp.float32),
        mesh=mesh,
        scratch_types=[
            pltpu.VMEM((CB,), jnp.int32),
            pltpu.VMEM((CB, 16), jnp.float32),
            pltpu.VMEM((CB, 16), jnp.float32),
            pltpu.SemaphoreType.DMA,
            pltpu.SemaphoreType.DMA,
        ],
    )
    def gather_kernel(nt_hbm, recv_hbm, send_hbm, g_hbm, idx_v, rows_r,
                      rows_s, sem_r, sem_s):
        c = lax.axis_index("c")
        s = lax.axis_index("s")
        wid = s * NC + c

        def chunk(k, carry):
            base = wid * epw + k * CB
            pltpu.sync_copy(recv_hbm.at[pl.ds(base, CB)], idx_v)
            pltpu.async_copy(nt_hbm.at[idx_v], rows_r, sem_r).wait()
            pltpu.sync_copy(send_hbm.at[pl.ds(base, CB)], idx_v)
            pltpu.async_copy(nt_hbm.at[idx_v], rows_s, sem_s).wait()
            pltpu.sync_copy(rows_r, g_hbm.at[pl.ds(base, CB), pl.ds(0, 16)])
            pltpu.sync_copy(rows_s, g_hbm.at[pl.ds(base, CB), pl.ds(16, 16)])
            return carry

        lax.fori_loop(0, nchunks, chunk, 0)

    return gather_kernel


def _zero_acc(acc, zbuf, s, n):
    stripe = n // NS
    piece = 625

    def zrow(i, carry):
        zbuf[i, pl.ds(0, 16)] = jnp.zeros((16,), jnp.float32)
        zbuf[i, pl.ds(16, 16)] = jnp.zeros((16,), jnp.float32)
        return carry
    lax.fori_loop(0, piece, zrow, 0)

    def zcopy(j, carry):
        pltpu.sync_copy(zbuf.at[pl.ds(0, piece)],
                        acc.at[pl.ds(s * stripe + j * piece, piece)])
        return carry
    lax.fori_loop(0, stripe // piece, zcopy, 0)


def _make_scatter_kernel(n, e, with_gather):
    ept = e // NS
    nchunks = ept // CD
    stripe = n // NS
    mesh = plsc.VectorSubcoreMesh(core_axis_name="c", subcore_axis_name="s")

    scratch = [
        pltpu.VMEM_SHARED((n, HALF), jnp.float32),
        pltpu.VMEM((CD,), jnp.int32),
        pltpu.VMEM((CD,), jnp.int32),
        pltpu.VMEM((CD, HALF), jnp.float32),
        pltpu.SemaphoreType.DMA,
        pltpu.SemaphoreType.DMA,
    ]

    def body(z_hbm, recv_hbm, send_hbm, p1_hbm, p2_hbm, agg_hbm, acc, idxr_v,
             idxs_v, zbuf, sem1, sem2):
        c = lax.axis_index("c")
        s = lax.axis_index("s")

        _zero_acc(acc, zbuf, s, n)
        plsc.subcore_barrier()

        def chunk(k, carry):
            base = s * ept + k * CD
            pltpu.sync_copy(recv_hbm.at[pl.ds(base, CD)], idxr_v)
            pltpu.sync_copy(z_hbm.at[c, pl.ds(base, CD)], zbuf)
            if with_gather:
                pltpu.sync_copy(send_hbm.at[pl.ds(base, CD)], idxs_v)
                pltpu.async_copy(p1_hbm.at[c, idxr_v], zbuf, sem1,
                                 add=True).wait()
                pltpu.async_copy(p2_hbm.at[c, idxs_v], zbuf, sem2,
                                 add=True).wait()
            _relu_chunk(zbuf, CD)
            pltpu.sync_copy(zbuf, acc.at[idxr_v], add=True)
            return carry

        lax.fori_loop(0, nchunks, chunk, 0)
        plsc.subcore_barrier()
        pltpu.sync_copy(acc.at[pl.ds(s * stripe, stripe)],
                        agg_hbm.at[c, pl.ds(s * stripe, stripe)])

    if with_gather:
        def fn(z_hbm, recv_hbm, send_hbm, p1_hbm, p2_hbm, agg_hbm, acc,
               idxr_v, idxs_v, zbuf, sem1, sem2):
            body(z_hbm, recv_hbm, send_hbm, p1_hbm, p2_hbm, agg_hbm, acc,
                 idxr_v, idxs_v, zbuf, sem1, sem2)
    else:
        def fn(z_hbm, recv_hbm, agg_hbm, acc, idxr_v, idxs_v, zbuf,
               sem1, sem2):
            body(z_hbm, recv_hbm, None, None, None, agg_hbm, acc, idxr_v,
                 idxs_v, zbuf, sem1, sem2)

    return functools.partial(
        pl.kernel,
        out_type=jax.ShapeDtypeStruct((NC, n, HALF), jnp.float32),
        mesh=mesh,
        scratch_types=scratch,
    )(fn)


def kernel(state, attr, receivers, senders, Ra, pe_w0, pe_b0, pe_w1, pe_b1,
           re_w0, re_b0, re_w1, re_b1, re_w2, re_b2, rp_w, rp_b, pp_w, pp_b,
           fp_w0, fp_b0, fp_w1, fp_b1, fp_w2, fp_b2):
    n = state.shape[0]
    e = receivers.shape[0]
    f32 = jnp.float32

    nt = jnp.concatenate([attr, state, jnp.zeros((n, 3), f32)], axis=1)

    z3w = jnp.zeros((3, NF), f32)
    pe_w0p = jnp.concatenate([pe_w0[0:7], pe_w0[13:19], z3w], axis=0)
    w0rs = jnp.concatenate([re_w0[0:7], re_w0[26:32], z3w,
                            re_w0[13:20], re_w0[32:38], z3w], axis=0)
    w_ra = re_w0[38:39]

    wa = rp_w[0:NF]
    w_p1 = rp_w[NF:2 * NF]
    w_p2 = rp_w[2 * NF:3 * NF]
    pp_wa = pp_w[0:NF]
    pp_wb = pp_w[NF:2 * NF]

    r1 = lambda b: b.reshape(1, -1)

    nb = n // BN
    pec = pl.pallas_call(
        _node_encode_body,
        grid=(nb,),
        in_specs=[pl.BlockSpec((BN, 16), lambda i: (i, 0)),
                  _full_spec((16, NF)), _full_spec((1, NF)),
                  _full_spec((NF, NF)), _full_spec((1, NF)),
                  _full_spec((NF, NF)), _full_spec((1, NF))],
        out_specs=pl.BlockSpec((BN, NF), lambda i: (i, 0)),
        out_shape=jax.ShapeDtypeStruct((n, NF), f32),
    )(nt, pe_w0p, r1(pe_b0), pe_w1, r1(pe_b1), pp_wa, r1(pp_b))

    g32 = _make_gather_kernel(n, e)(nt, receivers, senders)

    eb = e // BE
    z3 = pl.pallas_call(
        _rel_encode_body,
        grid=(eb,),
        in_specs=[pl.BlockSpec((BE, 32), lambda i: (i, 0)),
                  pl.BlockSpec((BE, 1), lambda i: (i, 0)),
                  _full_spec((32, NF)), _full_spec((1, NF)),
                  _full_spec((1, NF)),
                  _full_spec((NF, NF)), _full_spec((1, NF)),
                  _full_spec((NF, NF)), _full_spec((1, NF)),
                  _full_spec((NF, HALF)), _full_spec((NF, HALF)),
                  _full_spec((1, HALF)), _full_spec((1, HALF))],
        out_specs=pl.BlockSpec((NC, BE, HALF), lambda i: (0, i, 0)),
        out_shape=jax.ShapeDtypeStruct((NC, e, HALF), f32),
    )(g32, Ra, w0rs, w_ra, r1(re_b0), re_w1, r1(re_b1), re_w2, r1(re_b2),
      wa[:, :HALF], wa[:, HALF:], r1(rp_b[:HALF]), r1(rp_b[HALF:]))

    agg1 = _make_scatter_kernel(n, e, with_gather=False)(z3, receivers)

    p13, p23 = pl.pallas_call(
        _prop_node_body,
        grid=(nb,),
        in_specs=[pl.BlockSpec((NC, BN, HALF), lambda i: (0, i, 0)),
                  pl.BlockSpec((BN, NF), lambda i: (i, 0)),
                  _full_spec((HALF, NF)), _full_spec((HALF, NF)),
                  _full_spec((NF, HALF)), _full_spec((NF, HALF)),
                  _full_spec((NF, HALF)), _full_spec((NF, HALF))],
        out_specs=[pl.BlockSpec((NC, BN, HALF), lambda i: (0, i, 0)),
                   pl.BlockSpec((NC, BN, HALF), lambda i: (0, i, 0))],
        out_shape=[jax.ShapeDtypeStruct((NC, n, HALF), f32),
                   jax.ShapeDtypeStruct((NC, n, HALF), f32)],
    )(agg1, pec, pp_wb[:HALF], pp_wb[HALF:],
      w_p1[:, :HALF], w_p1[:, HALF:], w_p2[:, :HALF], w_p2[:, HALF:])

    agg2 = _make_scatter_kernel(n, e, with_gather=True)(
        z3, receivers, senders, p13, p23)

    pred = pl.pallas_call(
        _final_body,
        grid=(nb,),
        in_specs=[pl.BlockSpec((NC, BN, HALF), lambda i: (0, i, 0)),
                  pl.BlockSpec((BN, NF), lambda i: (i, 0)),
                  _full_spec((HALF, NF)), _full_spec((HALF, NF)),
                  _full_spec((NF, NF)), _full_spec((1, NF)),
                  _full_spec((NF, NF)), _full_spec((1, NF)),
                  _full_spec((NF, 3)), _full_spec((1, 3))],
        out_specs=pl.BlockSpec((BN, 3), lambda i: (i, 0)),
        out_shape=jax.ShapeDtypeStruct((n, 3), f32),
    )(agg2, pec, pp_wb[:HALF], pp_wb[HALF:], fp_w0, r1(fp_b0),
      fp_w1, r1(fp_b1), fp_w2, r1(fp_b2))

    return pred


# speedup vs baseline: 9.3240x; 9.3240x over previous
import functools

import jax
import jax.numpy as jnp
from jax import lax
from jax.experimental import pallas as pl
from jax.experimental.pallas import tpu as pltpu
from jax.experimental.pallas import tpu_sc as plsc

NF = 64
HALF = 32
NC = 2
NS = 16
BN = 2000
BQ = 2000
CB = 2000
CD = 400

_SC_PARAMS = pltpu.CompilerParams(use_tc_tiling_on_sc=False)


def _full_spec(shape):
    return pl.BlockSpec(shape, lambda i: tuple(0 for _ in shape))


def _mm(a, b):
    return jnp.dot(a, b, preferred_element_type=jnp.float32)


def _node_encode_body(nt_ref, w0_ref, b0_ref, w1_ref, b1_ref, wpa_ref,
                      bpa_ref, out_ref):
    h = jnp.maximum(_mm(nt_ref[...], w0_ref[...]) + b0_ref[...], 0.0)
    pe = jnp.maximum(_mm(h, w1_ref[...]) + b1_ref[...], 0.0)
    out_ref[...] = _mm(pe, wpa_ref[...]) + bpa_ref[...]


def _rel_encode_body(rs_ref, raa_ref, rab_ref, wqa_ref, wqb_ref, wra2_ref,
                     b0d_ref, w1d_ref, b1d_ref, w2d_ref, b2d_ref, wad_ref,
                     rpbd_ref, za_ref, zb_ref):
    blk = rs_ref[...]
    wra2 = wra2_ref[...]
    for ra_ref, wq_ref, z_ref in ((raa_ref, wqa_ref, za_ref),
                                  (rab_ref, wqb_ref, zb_ref)):
        h = _mm(blk, wq_ref[...]) + _mm(ra_ref[...], wra2) + b0d_ref[...]
        h = jnp.maximum(h, 0.0)
        h = jnp.maximum(_mm(h, w1d_ref[...]) + b1d_ref[...], 0.0)
        rel = jnp.maximum(_mm(h, w2d_ref[...]) + b2d_ref[...], 0.0)
        z_ref[...] = _mm(rel, wad_ref[...]) + rpbd_ref[...]


def _prop_node_body(agg_ref, pec_ref, ppb_ref, wp12_ref, out_ref):
    pe = jnp.maximum(pec_ref[...] + _mm(agg_ref[...], ppb_ref[...]), 0.0)
    out_ref[...] = _mm(pe, wp12_ref[...])


def _final_body(agg_ref, pec_ref, ppb_ref, fw0_ref, fb0_ref, fw1_ref,
                fb1_ref, fw2_ref, fb2_ref, out_ref):
    pe = jnp.maximum(pec_ref[...] + _mm(agg_ref[...], ppb_ref[...]), 0.0)
    p = jnp.maximum(_mm(pe, fw0_ref[...]) + fb0_ref[...], 0.0)
    p = jnp.maximum(_mm(p, fw1_ref[...]) + fb1_ref[...], 0.0)
    out_ref[...] = _mm(p, fw2_ref[...]) + fb2_ref[...]


def _make_gather_kernel(n, e):
    eq = e // 4
    nchunks = eq // CB
    mesh = plsc.VectorSubcoreMesh(core_axis_name="c", subcore_axis_name="s")

    @functools.partial(
        pl.kernel,
        out_type=jax.ShapeDtypeStruct((eq, 128), jnp.float32),
        mesh=mesh,
        scratch_types=[
            pltpu.VMEM((CB,), jnp.int32),
            pltpu.VMEM((CB, 16), jnp.float32),
            pltpu.SemaphoreType.DMA,
        ],
        compiler_params=_SC_PARAMS,
    )
    def gather_kernel(nt_hbm, recv_hbm, send_hbm, rs_hbm, idx_v, rows_v, sem):
        c = lax.axis_index("c")
        s = lax.axis_index("s")
        wid = s * NC + c

        def rounds(k, carry):
            ch = wid + k * (NC * NS)

            @pl.when(ch < nchunks)
            def _():
                base = ch * CB
                for q in range(4):
                    for ep, col in ((recv_hbm, q * 32), (send_hbm, q * 32 + 16)):
                        pltpu.sync_copy(ep.at[pl.ds(q * eq + base, CB)], idx_v)
                        pltpu.async_copy(nt_hbm.at[idx_v], rows_v, sem).wait()
                        pltpu.sync_copy(
                            rows_v, rs_hbm.at[pl.ds(base, CB), pl.ds(col, 16)])
            return carry

        nrounds = (nchunks + NC * NS - 1) // (NC * NS)
        lax.fori_loop(0, nrounds, rounds, 0)

    return gather_kernel


def _relu_chunk(buf, rows):
    def body(i, carry):
        a = buf[i, pl.ds(0, 16)]
        buf[i, pl.ds(0, 16)] = jnp.maximum(a, 0.0)
        b = buf[i, pl.ds(16, 16)]
        buf[i, pl.ds(16, 16)] = jnp.maximum(b, 0.0)
        return carry
    lax.fori_loop(0, rows, body, 0)


def _zero_rows(buf, rows):
    def body(i, carry):
        buf[i, pl.ds(0, 16)] = jnp.zeros((16,), jnp.float32)
        buf[i, pl.ds(16, 16)] = jnp.zeros((16,), jnp.float32)
        return carry
    lax.fori_loop(0, rows, body, 0)


def _make_scatter_kernel(n, e, with_gather):
    eq = e // 4
    ept = eq // 4
    nchunks = ept // CD
    stripe = n // NS
    stripe_pieces = [(r0, min(CD, stripe - r0)) for r0 in range(0, stripe, CD)]
    mesh = plsc.VectorSubcoreMesh(core_axis_name="c", subcore_axis_name="s")

    scratch = [
        pltpu.VMEM_SHARED((n, HALF), jnp.float32),
        pltpu.VMEM((CD,), jnp.int32),
        pltpu.VMEM((CD,), jnp.int32),
        pltpu.VMEM((CD,), jnp.int32),
        pltpu.VMEM((CD, HALF), jnp.float32),
        pltpu.SemaphoreType.DMA,
    ]
    if with_gather:
        out_type = [jax.ShapeDtypeStruct((n, 128), jnp.float32),
                    jax.ShapeDtypeStruct((NC * n, HALF), jnp.float32),
                    jax.ShapeDtypeStruct((NC * n, HALF), jnp.float32)]
    else:
        out_type = jax.ShapeDtypeStruct((n, 128), jnp.float32)

    def body(za_hbm, zb_hbm, recv_hbm, send_hbm, p12_hbm, agg_hbm, p1f_hbm,
             p2f_hbm, acc, idxr_v, idxrg_v, idxs_v, zbuf, sem):
        c = lax.axis_index("c")
        s = lax.axis_index("s")
        cn = (c * n).astype(jnp.int32)

        _zero_rows(zbuf, CD)
        for r0, rcnt in stripe_pieces:
            pltpu.sync_copy(zbuf.at[pl.ds(0, rcnt)],
                            acc.at[pl.ds(s * stripe + r0, rcnt)])

        if with_gather:
            for src_col, dst in ((c * HALF, p1f_hbm),
                                 (NF + c * HALF, p2f_hbm)):
                for r0, rcnt in stripe_pieces:
                    pltpu.sync_copy(
                        p12_hbm.at[pl.ds(s * stripe + r0, rcnt),
                                   pl.ds(src_col, HALF)],
                        zbuf.at[pl.ds(0, rcnt)])
                    pltpu.sync_copy(
                        zbuf.at[pl.ds(0, rcnt)],
                        dst.at[pl.ds(cn + s * stripe + r0, rcnt)])

        plsc.subcore_barrier()

        def stream_loop(z_hbm, qhalf, idx_ofs, row0):
            colbase = qhalf * NF + c * HALF

            def chunk(k, carry):
                base = row0 + k * CD
                pltpu.sync_copy(recv_hbm.at[pl.ds(idx_ofs + base, CD)], idxr_v)
                pltpu.sync_copy(z_hbm.at[pl.ds(base, CD),
                                         pl.ds(colbase, HALF)], zbuf)
                if with_gather:
                    pltpu.sync_copy(send_hbm.at[pl.ds(idx_ofs + base, CD)],
                                    idxs_v)

                    def ofs(j, carry2):
                        idxrg_v[pl.ds(j * 16, 16)] = (
                            idxr_v[pl.ds(j * 16, 16)] + cn)
                        idxs_v[pl.ds(j * 16, 16)] = (
                            idxs_v[pl.ds(j * 16, 16)] + cn)
                        return carry2
                    lax.fori_loop(0, CD // 16, ofs, 0)

                    pltpu.async_copy(p1f_hbm.at[idxrg_v], zbuf, sem,
                                     add=True).wait()
                    pltpu.async_copy(p2f_hbm.at[idxs_v], zbuf, sem,
                                     add=True).wait()
                _relu_chunk(zbuf, CD)
                pltpu.sync_copy(zbuf, acc.at[idxr_v], add=True)
                return carry

            lax.fori_loop(0, nchunks, chunk, 0)

        sid = s // 4
        row0 = (s % 4) * ept
        for q, (zref, qhalf) in enumerate(((za_hbm, 0), (za_hbm, 1),
                                           (zb_hbm, 0), (zb_hbm, 1))):
            @pl.when(sid == q)
            def _(zref=zref, qhalf=qhalf, q=q):
                stream_loop(zref, qhalf, q * eq, row0)

        plsc.subcore_barrier()

        pltpu.sync_copy(acc.at[pl.ds(s * stripe, stripe)],
                        agg_hbm.at[pl.ds(s * stripe, stripe),
                                   pl.ds(c * HALF, HALF)])
        _zero_rows(zbuf, CD)
        for r0, rcnt in stripe_pieces:
            pltpu.sync_copy(
                zbuf.at[pl.ds(0, rcnt)],
                agg_hbm.at[pl.ds(s * stripe + r0, rcnt),
                           pl.ds(NF + c * HALF, HALF)])

    if with_gather:
        def fn(za_hbm, zb_hbm, recv_hbm, send_hbm, p12_hbm, agg_hbm, p1f_hbm,
               p2f_hbm, acc, idxr_v, idxrg_v, idxs_v, zbuf, sem):
            body(za_hbm, zb_hbm, recv_hbm, send_hbm, p12_hbm, agg_hbm,
                 p1f_hbm, p2f_hbm, acc, idxr_v, idxrg_v, idxs_v, zbuf, sem)
    else:
        def fn(za_hbm, zb_hbm, recv_hbm, agg_hbm, acc, idxr_v, idxrg_v,
               idxs_v, zbuf, sem):
            body(za_hbm, zb_hbm, recv_hbm, None, None, agg_hbm, None, None,
                 acc, idxr_v, idxrg_v, idxs_v, zbuf, sem)

    return functools.partial(
        pl.kernel,
        out_type=out_type,
        mesh=mesh,
        scratch_types=scratch,
        compiler_params=_SC_PARAMS,
    )(fn)


def kernel(state, attr, receivers, senders, Ra, pe_w0, pe_b0, pe_w1, pe_b1,
           re_w0, re_b0, re_w1, re_b1, re_w2, re_b2, rp_w, rp_b, pp_w, pp_b,
           fp_w0, fp_b0, fp_w1, fp_b1, fp_w2, fp_b2):
    n = state.shape[0]
    e = receivers.shape[0]
    eq = e // 4
    f32 = jnp.float32
    r1 = lambda b: b.reshape(1, -1)
    zz = lambda r, c: jnp.zeros((r, c), f32)

    nt = jnp.concatenate([attr, state, zz(n, 3)], axis=1)

    pe_w0p = jnp.concatenate([pe_w0[0:7], pe_w0[13:19], zz(3, NF)], axis=0)
    w0r = jnp.concatenate([re_w0[0:7], re_w0[26:32], zz(3, NF)], axis=0)
    w0s = jnp.concatenate([re_w0[13:20], re_w0[32:38], zz(3, NF)], axis=0)
    w_ra = re_w0[38:39]

    def dup(w):
        k = w.shape[0]
        return jnp.concatenate(
            [jnp.concatenate([w, zz(k, NF)], 1),
             jnp.concatenate([zz(k, NF), w], 1)], 0)

    wqa = jnp.concatenate([dup1 for dup1 in (
        jnp.concatenate([w0r, zz(16, NF)], 1),
        jnp.concatenate([w0s, zz(16, NF)], 1),
        jnp.concatenate([zz(16, NF), w0r], 1),
        jnp.concatenate([zz(16, NF), w0s], 1),
        zz(64, 2 * NF))], axis=0)
    wqb = jnp.concatenate([
        zz(64, 2 * NF),
        jnp.concatenate([w0r, zz(16, NF)], 1),
        jnp.concatenate([w0s, zz(16, NF)], 1),
        jnp.concatenate([zz(16, NF), w0r], 1),
        jnp.concatenate([zz(16, NF), w0s], 1)], axis=0)
    wra2 = dup(w_ra)
    w1d, w2d = dup(re_w1), dup(re_w2)
    wad = dup(rp_w[0:NF])
    b0d = r1(jnp.concatenate([re_b0, re_b0]))
    b1d = r1(jnp.concatenate([re_b1, re_b1]))
    b2d = r1(jnp.concatenate([re_b2, re_b2]))
    rpbd = r1(jnp.concatenate([rp_b, rp_b]))

    w_p12 = jnp.concatenate([rp_w[NF:2 * NF], rp_w[2 * NF:3 * NF]], axis=1)
    ppb128 = jnp.concatenate([pp_w[NF:2 * NF], zz(NF, NF)], axis=0)
    ra_a = jnp.concatenate([Ra[0:eq], Ra[eq:2 * eq]], axis=1)
    ra_b = jnp.concatenate([Ra[2 * eq:3 * eq], Ra[3 * eq:]], axis=1)

    nb = n // BN
    pec = pl.pallas_call(
        _node_encode_body,
        grid=(nb,),
        in_specs=[pl.BlockSpec((BN, 16), lambda i: (i, 0)),
                  _full_spec((16, NF)), _full_spec((1, NF)),
                  _full_spec((NF, NF)), _full_spec((1, NF)),
                  _full_spec((NF, NF)), _full_spec((1, NF))],
        out_specs=pl.BlockSpec((BN, NF), lambda i: (i, 0)),
        out_shape=jax.ShapeDtypeStruct((n, NF), f32),
    )(nt, pe_w0p, r1(pe_b0), pe_w1, r1(pe_b1), pp_w[0:NF], r1(pp_b))

    rs = _make_gather_kernel(n, e)(nt, receivers, senders)

    z_a, z_b = pl.pallas_call(
        _rel_encode_body,
        grid=(eq // BQ,),
        in_specs=[pl.BlockSpec((BQ, 128), lambda i: (i, 0)),
                  pl.BlockSpec((BQ, 2), lambda i: (i, 0)),
                  pl.BlockSpec((BQ, 2), lambda i: (i, 0)),
                  _full_spec((128, 128)), _full_spec((128, 128)),
                  _full_spec((2, 128)), _full_spec((1, 128)),
                  _full_spec((128, 128)), _full_spec((1, 128)),
                  _full_spec((128, 128)), _full_spec((1, 128)),
                  _full_spec((128, 128)), _full_spec((1, 128))],
        out_specs=[pl.BlockSpec((BQ, 128), lambda i: (i, 0)),
                   pl.BlockSpec((BQ, 128), lambda i: (i, 0))],
        out_shape=[jax.ShapeDtypeStruct((eq, 128), f32),
                   jax.ShapeDtypeStruct((eq, 128), f32)],
    )(rs, ra_a, ra_b, wqa, wqb, wra2, b0d, w1d, b1d, w2d, b2d, wad, rpbd)

    agg1 = _make_scatter_kernel(n, e, with_gather=False)(
        z_a, z_b, receivers)

    p12 = pl.pallas_call(
        _prop_node_body,
        grid=(nb,),
        in_specs=[pl.BlockSpec((BN, 128), lambda i: (i, 0)),
                  pl.BlockSpec((BN, NF), lambda i: (i, 0)),
                  _full_spec((128, NF)), _full_spec((NF, 128))],
        out_specs=pl.BlockSpec((BN, 128), lambda i: (i, 0)),
        out_shape=jax.ShapeDtypeStruct((n, 128), f32),
    )(agg1, pec, ppb128, w_p12)

    agg2, _, _ = _make_scatter_kernel(n, e, with_gather=True)(
        z_a, z_b, receivers, senders, p12)

    pred = pl.pallas_call(
        _final_body,
        grid=(nb,),
        in_specs=[pl.BlockSpec((BN, 128), lambda i: (i, 0)),
                  pl.BlockSpec((BN, NF), lambda i: (i, 0)),
                  _full_spec((128, NF)),
                  _full_spec((NF, NF)), _full_spec((1, NF)),
                  _full_spec((NF, NF)), _full_spec((1, NF)),
                  _full_spec((NF, 3)), _full_spec((1, 3))],
        out_specs=pl.BlockSpec((BN, 3), lambda i: (i, 0)),
        out_shape=jax.ShapeDtypeStruct((n, 3), f32),
    )(agg2, pec, ppb128, fp_w0, r1(fp_b0), fp_w1, r1(fp_b1),
      fp_w2, r1(fp_b2))

    return pred


# speedup vs baseline: 11.2991x; 1.2118x over previous
import functools

import jax
import jax.numpy as jnp
from jax import lax
from jax.experimental import pallas as pl
from jax.experimental.pallas import tpu as pltpu
from jax.experimental.pallas import tpu_sc as plsc

NF = 64
HALF = 32
NC = 2
NS = 16
BN = 2000
BQ = 2000
CB = 2000
CD = 400

_SC_PARAMS = pltpu.CompilerParams(use_tc_tiling_on_sc=False)


def _full_spec(shape):
    return pl.BlockSpec(shape, lambda i: tuple(0 for _ in shape))


def _mm(a, b):
    return jnp.dot(a, b, preferred_element_type=jnp.float32)


def _node_encode_body(nt_ref, w0_ref, b0_ref, w1_ref, b1_ref, wpa_ref,
                      bpa_ref, out_ref):
    h = jnp.maximum(_mm(nt_ref[...], w0_ref[...]) + b0_ref[...], 0.0)
    pe = jnp.maximum(_mm(h, w1_ref[...]) + b1_ref[...], 0.0)
    out_ref[...] = _mm(pe, wpa_ref[...]) + bpa_ref[...]


def _rel_encode_body(rs_ref, raa_ref, rab_ref, wqa_ref, wqb_ref, wra2_ref,
                     b0d_ref, w1d_ref, b1d_ref, w2d_ref, b2d_ref, wad_ref,
                     rpbd_ref, za_ref, zb_ref, zra_ref, zrb_ref):
    blk = rs_ref[...]
    wra2 = wra2_ref[...]
    for ra_ref, wq_ref, z_ref, zr_ref in ((raa_ref, wqa_ref, za_ref, zra_ref),
                                          (rab_ref, wqb_ref, zb_ref, zrb_ref)):
        h = _mm(blk, wq_ref[...]) + _mm(ra_ref[...], wra2) + b0d_ref[...]
        h = jnp.maximum(h, 0.0)
        h = jnp.maximum(_mm(h, w1d_ref[...]) + b1d_ref[...], 0.0)
        rel = jnp.maximum(_mm(h, w2d_ref[...]) + b2d_ref[...], 0.0)
        z = _mm(rel, wad_ref[...]) + rpbd_ref[...]
        z_ref[...] = z
        zr_ref[...] = jnp.maximum(z, 0.0)


def _prop_node_body(agg_ref, pec_ref, ppb_ref, wp12_ref, out_ref):
    pe = jnp.maximum(pec_ref[...] + _mm(agg_ref[...], ppb_ref[...]), 0.0)
    out_ref[...] = _mm(pe, wp12_ref[...])


def _final_body(agg_ref, pec_ref, ppb_ref, fw0_ref, fb0_ref, fw1_ref,
                fb1_ref, fw2_ref, fb2_ref, out_ref):
    pe = jnp.maximum(pec_ref[...] + _mm(agg_ref[...], ppb_ref[...]), 0.0)
    p = jnp.maximum(_mm(pe, fw0_ref[...]) + fb0_ref[...], 0.0)
    p = jnp.maximum(_mm(p, fw1_ref[...]) + fb1_ref[...], 0.0)
    out_ref[...] = _mm(p, fw2_ref[...]) + fb2_ref[...]


def _make_gather_kernel(n, e):
    eq = e // 4
    nchunks = eq // CB
    mesh = plsc.VectorSubcoreMesh(core_axis_name="c", subcore_axis_name="s")

    @functools.partial(
        pl.kernel,
        out_type=jax.ShapeDtypeStruct((eq, 128), jnp.float32),
        mesh=mesh,
        scratch_types=[
            pltpu.VMEM((CB,), jnp.int32),
            pltpu.VMEM((CB, 16), jnp.float32),
            pltpu.SemaphoreType.DMA,
        ],
        compiler_params=_SC_PARAMS,
    )
    def gather_kernel(nt_hbm, recv_hbm, send_hbm, rs_hbm, idx_v, rows_v, sem):
        c = lax.axis_index("c")
        s = lax.axis_index("s")
        wid = s * NC + c

        def rounds(k, carry):
            ch = wid + k * (NC * NS)

            @pl.when(ch < nchunks)
            def _():
                base = ch * CB
                for q in range(4):
                    for ep, col in ((recv_hbm, q * 32), (send_hbm, q * 32 + 16)):
                        pltpu.sync_copy(ep.at[pl.ds(q * eq + base, CB)], idx_v)
                        pltpu.async_copy(nt_hbm.at[idx_v], rows_v, sem).wait()
                        pltpu.sync_copy(
                            rows_v, rs_hbm.at[pl.ds(base, CB), pl.ds(col, 16)])
            return carry

        nrounds = (nchunks + NC * NS - 1) // (NC * NS)
        lax.fori_loop(0, nrounds, rounds, 0)

    return gather_kernel


def _add_relu_chunk(buf, gbuf, rows):
    def body(i, carry):
        for r in range(2):
            for col in (0, 16):
                sl = (2 * i + r, pl.ds(col, 16))
                buf[sl] = jnp.maximum(buf[sl] + gbuf[sl], 0.0)
        return carry
    lax.fori_loop(0, rows // 2, body, 0)


def _zero_rows(buf, rows):
    def body(i, carry):
        buf[i, pl.ds(0, 16)] = jnp.zeros((16,), jnp.float32)
        buf[i, pl.ds(16, 16)] = jnp.zeros((16,), jnp.float32)
        return carry
    lax.fori_loop(0, rows, body, 0)


def _agg_writeout(acc, agg_hbm, zbuf, c, s, stripe, stripe_pieces):
    pltpu.sync_copy(acc.at[pl.ds(s * stripe, stripe)],
                    agg_hbm.at[pl.ds(s * stripe, stripe),
                               pl.ds(c * HALF, HALF)])
    _zero_rows(zbuf, min(stripe_pieces[0][1], stripe))
    for r0, rcnt in stripe_pieces:
        pltpu.sync_copy(
            zbuf.at[pl.ds(0, rcnt)],
            agg_hbm.at[pl.ds(s * stripe + r0, rcnt),
                       pl.ds(NF + c * HALF, HALF)])


def _make_scatter1_kernel(n, e):
    eq = e // 4
    cd = 800
    nchunks = eq // cd
    nrounds = (nchunks + 3) // 4
    stripe = n // NS
    stripe_pieces = [(r0, min(cd, stripe - r0)) for r0 in range(0, stripe, cd)]
    mesh = plsc.VectorSubcoreMesh(core_axis_name="c", subcore_axis_name="s")

    @functools.partial(
        pl.kernel,
        out_type=jax.ShapeDtypeStruct((n, 128), jnp.float32),
        mesh=mesh,
        scratch_types=[
            pltpu.VMEM_SHARED((n, HALF), jnp.float32),
            pltpu.VMEM((cd,), jnp.int32),
            pltpu.VMEM((cd, HALF), jnp.float32),
        ],
        compiler_params=_SC_PARAMS,
    )
    def fn(za_hbm, zb_hbm, recv_hbm, agg_hbm, acc, idxr_v, zbuf):
        c = lax.axis_index("c")
        s = lax.axis_index("s")

        _zero_rows(zbuf, min(cd, stripe))
        for r0, rcnt in stripe_pieces:
            pltpu.sync_copy(zbuf.at[pl.ds(0, rcnt)],
                            acc.at[pl.ds(s * stripe + r0, rcnt)])
        plsc.subcore_barrier()

        sid = s // 4
        lane = s % 4

        def stream_loop(z_hbm, qhalf, idx_ofs):
            colbase = qhalf * NF + c * HALF

            def rounds(k, carry):
                ch = lane + 4 * k

                @pl.when(ch < nchunks)
                def _():
                    base = ch * cd
                    pltpu.sync_copy(recv_hbm.at[pl.ds(idx_ofs + base, cd)],
                                    idxr_v)
                    pltpu.sync_copy(z_hbm.at[pl.ds(base, cd),
                                             pl.ds(colbase, HALF)], zbuf)
                    pltpu.sync_copy(zbuf, acc.at[idxr_v], add=True)
                return carry

            lax.fori_loop(0, nrounds, rounds, 0)

        for q, (zref, qhalf) in enumerate(((za_hbm, 0), (za_hbm, 1),
                                           (zb_hbm, 0), (zb_hbm, 1))):
            @pl.when(sid == q)
            def _(zref=zref, qhalf=qhalf, q=q):
                stream_loop(zref, qhalf, q * eq)

        plsc.subcore_barrier()
        _agg_writeout(acc, agg_hbm, zbuf, c, s, stripe, stripe_pieces)

    return fn


def _make_scatter2_kernel(n, e):
    eq = e // 4
    ept = eq // 4
    nchunks = ept // CD
    stripe = n // NS
    stripe_pieces = [(r0, min(CD, stripe - r0)) for r0 in range(0, stripe, CD)]
    mesh = plsc.VectorSubcoreMesh(core_axis_name="c", subcore_axis_name="s")

    @functools.partial(
        pl.kernel,
        out_type=[jax.ShapeDtypeStruct((n, 128), jnp.float32),
                  jax.ShapeDtypeStruct((NC * n, HALF), jnp.float32),
                  jax.ShapeDtypeStruct((NC * n, HALF), jnp.float32)],
        mesh=mesh,
        scratch_types=[
            pltpu.VMEM_SHARED((n, HALF), jnp.float32),
            pltpu.VMEM((CD,), jnp.int32),
            pltpu.VMEM((CD,), jnp.int32),
            pltpu.VMEM((CD,), jnp.int32),
            pltpu.VMEM((CD, HALF), jnp.float32),
            pltpu.VMEM((CD, HALF), jnp.float32),
            pltpu.SemaphoreType.DMA,
            pltpu.SemaphoreType.DMA,
        ],
        compiler_params=_SC_PARAMS,
    )
    def fn(za_hbm, zb_hbm, recv_hbm, send_hbm, p12_hbm, agg_hbm, p1f_hbm,
           p2f_hbm, acc, idxr_v, idxrg_v, idxs_v, zbuf, gbuf, sem1, sem2):
        c = lax.axis_index("c")
        s = lax.axis_index("s")
        cn = (c * n).astype(jnp.int32)

        _zero_rows(zbuf, min(CD, stripe))
        for r0, rcnt in stripe_pieces:
            pltpu.sync_copy(zbuf.at[pl.ds(0, rcnt)],
                            acc.at[pl.ds(s * stripe + r0, rcnt)])

        for src_col, dst in ((c * HALF, p1f_hbm), (NF + c * HALF, p2f_hbm)):
            for r0, rcnt in stripe_pieces:
                pltpu.sync_copy(
                    p12_hbm.at[pl.ds(s * stripe + r0, rcnt),
                               pl.ds(src_col, HALF)],
                    zbuf.at[pl.ds(0, rcnt)])
                pltpu.sync_copy(zbuf.at[pl.ds(0, rcnt)],
                                dst.at[pl.ds(cn + s * stripe + r0, rcnt)])

        plsc.subcore_barrier()

        def stream_loop(z_hbm, qhalf, idx_ofs, row0):
            colbase = qhalf * NF + c * HALF

            def chunk(k, carry):
                base = row0 + k * CD
                pltpu.sync_copy(recv_hbm.at[pl.ds(idx_ofs + base, CD)], idxr_v)
                pltpu.sync_copy(send_hbm.at[pl.ds(idx_ofs + base, CD)], idxs_v)
                pltpu.sync_copy(z_hbm.at[pl.ds(base, CD),
                                         pl.ds(colbase, HALF)], zbuf)

                def ofs(j, carry2):
                    idxrg_v[pl.ds(j * 16, 16)] = idxr_v[pl.ds(j * 16, 16)] + cn
                    idxs_v[pl.ds(j * 16, 16)] = idxs_v[pl.ds(j * 16, 16)] + cn
                    return carry2
                lax.fori_loop(0, CD // 16, ofs, 0)

                cp1 = pltpu.async_copy(p1f_hbm.at[idxrg_v], zbuf, sem1,
                                       add=True)
                cp2 = pltpu.async_copy(p2f_hbm.at[idxs_v], gbuf, sem2)
                cp1.wait()
                cp2.wait()
                _add_relu_chunk(zbuf, gbuf, CD)
                pltpu.sync_copy(zbuf, acc.at[idxr_v], add=True)
                return carry

            lax.fori_loop(0, nchunks, chunk, 0)

        sid = s // 4
        row0 = (s % 4) * ept
        for q, (zref, qhalf) in enumerate(((za_hbm, 0), (za_hbm, 1),
                                           (zb_hbm, 0), (zb_hbm, 1))):
            @pl.when(sid == q)
            def _(zref=zref, qhalf=qhalf, q=q):
                stream_loop(zref, qhalf, q * eq, row0)

        plsc.subcore_barrier()
        _agg_writeout(acc, agg_hbm, zbuf, c, s, stripe, stripe_pieces)

    return fn


def kernel(state, attr, receivers, senders, Ra, pe_w0, pe_b0, pe_w1, pe_b1,
           re_w0, re_b0, re_w1, re_b1, re_w2, re_b2, rp_w, rp_b, pp_w, pp_b,
           fp_w0, fp_b0, fp_w1, fp_b1, fp_w2, fp_b2):
    n = state.shape[0]
    e = receivers.shape[0]
    eq = e // 4
    f32 = jnp.float32
    r1 = lambda b: b.reshape(1, -1)
    zz = lambda r, c: jnp.zeros((r, c), f32)

    nt = jnp.concatenate([attr, state, zz(n, 3)], axis=1)

    pe_w0p = jnp.concatenate([pe_w0[0:7], pe_w0[13:19], zz(3, NF)], axis=0)
    w0r = jnp.concatenate([re_w0[0:7], re_w0[26:32], zz(3, NF)], axis=0)
    w0s = jnp.concatenate([re_w0[13:20], re_w0[32:38], zz(3, NF)], axis=0)
    w_ra = re_w0[38:39]

    def dup(w):
        k = w.shape[0]
        return jnp.concatenate(
            [jnp.concatenate([w, zz(k, NF)], 1),
             jnp.concatenate([zz(k, NF), w], 1)], 0)

    wqa = jnp.concatenate([dup1 for dup1 in (
        jnp.concatenate([w0r, zz(16, NF)], 1),
        jnp.concatenate([w0s, zz(16, NF)], 1),
        jnp.concatenate([zz(16, NF), w0r], 1),
        jnp.concatenate([zz(16, NF), w0s], 1),
        zz(64, 2 * NF))], axis=0)
    wqb = jnp.concatenate([
        zz(64, 2 * NF),
        jnp.concatenate([w0r, zz(16, NF)], 1),
        jnp.concatenate([w0s, zz(16, NF)], 1),
        jnp.concatenate([zz(16, NF), w0r], 1),
        jnp.concatenate([zz(16, NF), w0s], 1)], axis=0)
    wra2 = dup(w_ra)
    w1d, w2d = dup(re_w1), dup(re_w2)
    wad = dup(rp_w[0:NF])
    b0d = r1(jnp.concatenate([re_b0, re_b0]))
    b1d = r1(jnp.concatenate([re_b1, re_b1]))
    b2d = r1(jnp.concatenate([re_b2, re_b2]))
    rpbd = r1(jnp.concatenate([rp_b, rp_b]))

    w_p12 = jnp.concatenate([rp_w[NF:2 * NF], rp_w[2 * NF:3 * NF]], axis=1)
    ppb128 = jnp.concatenate([pp_w[NF:2 * NF], zz(NF, NF)], axis=0)
    ra_a = jnp.concatenate([Ra[0:eq], Ra[eq:2 * eq]], axis=1)
    ra_b = jnp.concatenate([Ra[2 * eq:3 * eq], Ra[3 * eq:]], axis=1)

    nb = n // BN
    pec = pl.pallas_call(
        _node_encode_body,
        grid=(nb,),
        in_specs=[pl.BlockSpec((BN, 16), lambda i: (i, 0)),
                  _full_spec((16, NF)), _full_spec((1, NF)),
                  _full_spec((NF, NF)), _full_spec((1, NF)),
                  _full_spec((NF, NF)), _full_spec((1, NF))],
        out_specs=pl.BlockSpec((BN, NF), lambda i: (i, 0)),
        out_shape=jax.ShapeDtypeStruct((n, NF), f32),
    )(nt, pe_w0p, r1(pe_b0), pe_w1, r1(pe_b1), pp_w[0:NF], r1(pp_b))

    rs = _make_gather_kernel(n, e)(nt, receivers, senders)

    z_a, z_b, zr_a, zr_b = pl.pallas_call(
        _rel_encode_body,
        grid=(eq // BQ,),
        in_specs=[pl.BlockSpec((BQ, 128), lambda i: (i, 0)),
                  pl.BlockSpec((BQ, 2), lambda i: (i, 0)),
                  pl.BlockSpec((BQ, 2), lambda i: (i, 0)),
                  _full_spec((128, 128)), _full_spec((128, 128)),
                  _full_spec((2, 128)), _full_spec((1, 128)),
                  _full_spec((128, 128)), _full_spec((1, 128)),
                  _full_spec((128, 128)), _full_spec((1, 128)),
                  _full_spec((128, 128)), _full_spec((1, 128))],
        out_specs=[pl.BlockSpec((BQ, 128), lambda i: (i, 0)),
                   pl.BlockSpec((BQ, 128), lambda i: (i, 0)),
                   pl.BlockSpec((BQ, 128), lambda i: (i, 0)),
                   pl.BlockSpec((BQ, 128), lambda i: (i, 0))],
        out_shape=[jax.ShapeDtypeStruct((eq, 128), f32),
                   jax.ShapeDtypeStruct((eq, 128), f32),
                   jax.ShapeDtypeStruct((eq, 128), f32),
                   jax.ShapeDtypeStruct((eq, 128), f32)],
    )(rs, ra_a, ra_b, wqa, wqb, wra2, b0d, w1d, b1d, w2d, b2d, wad, rpbd)

    agg1 = _make_scatter1_kernel(n, e)(zr_a, zr_b, receivers)

    p12 = pl.pallas_call(
        _prop_node_body,
        grid=(nb,),
        in_specs=[pl.BlockSpec((BN, 128), lambda i: (i, 0)),
                  pl.BlockSpec((BN, NF), lambda i: (i, 0)),
                  _full_spec((128, NF)), _full_spec((NF, 128))],
        out_specs=pl.BlockSpec((BN, 128), lambda i: (i, 0)),
        out_shape=jax.ShapeDtypeStruct((n, 128), f32),
    )(agg1, pec, ppb128, w_p12)

    agg2, _, _ = _make_scatter2_kernel(n, e)(
        z_a, z_b, receivers, senders, p12)

    pred = pl.pallas_call(
        _final_body,
        grid=(nb,),
        in_specs=[pl.BlockSpec((BN, 128), lambda i: (i, 0)),
                  pl.BlockSpec((BN, NF), lambda i: (i, 0)),
                  _full_spec((128, NF)),
                  _full_spec((NF, NF)), _full_spec((1, NF)),
                  _full_spec((NF, NF)), _full_spec((1, NF)),
                  _full_spec((NF, 3)), _full_spec((1, 3))],
        out_specs=pl.BlockSpec((BN, 3), lambda i: (i, 0)),
        out_shape=jax.ShapeDtypeStruct((n, 3), f32),
    )(agg2, pec, ppb128, fp_w0, r1(fp_b0), fp_w1, r1(fp_b1),
      fp_w2, r1(fp_b2))

    return pred


# speedup vs baseline: 12.1905x; 1.0789x over previous
import functools

import jax
import jax.numpy as jnp
from jax import lax
from jax.experimental import pallas as pl
from jax.experimental.pallas import tpu as pltpu
from jax.experimental.pallas import tpu_sc as plsc

NF = 64
HALF = 32
NC = 2
NS = 16
BN = 2000
BQ = 2000
CB = 2000
CD = 200

_SC_PARAMS = pltpu.CompilerParams(use_tc_tiling_on_sc=False)


def _full_spec(shape):
    return pl.BlockSpec(shape, lambda i: tuple(0 for _ in shape))


def _mm(a, b):
    return jnp.dot(a, b, preferred_element_type=jnp.float32)


def _node_encode_body(nt_ref, w0_ref, b0_ref, w1_ref, b1_ref, wpa_ref,
                      bpa_ref, out_ref):
    h = jnp.maximum(_mm(nt_ref[...], w0_ref[...]) + b0_ref[...], 0.0)
    pe = jnp.maximum(_mm(h, w1_ref[...]) + b1_ref[...], 0.0)
    out_ref[...] = _mm(pe, wpa_ref[...]) + bpa_ref[...]


def _rel_encode_body(rs_ref, raa_ref, rab_ref, wqa_ref, wqb_ref, wra2_ref,
                     b0d_ref, w1d_ref, b1d_ref, w2d_ref, b2d_ref, wad_ref,
                     rpbd_ref, za_ref, zb_ref, zra_ref, zrb_ref):
    blk = rs_ref[...]
    wra2 = wra2_ref[...]
    for ra_ref, wq_ref, z_ref, zr_ref in ((raa_ref, wqa_ref, za_ref, zra_ref),
                                          (rab_ref, wqb_ref, zb_ref, zrb_ref)):
        h = _mm(blk, wq_ref[...]) + _mm(ra_ref[...], wra2) + b0d_ref[...]
        h = jnp.maximum(h, 0.0)
        h = jnp.maximum(_mm(h, w1d_ref[...]) + b1d_ref[...], 0.0)
        rel = jnp.maximum(_mm(h, w2d_ref[...]) + b2d_ref[...], 0.0)
        z = _mm(rel, wad_ref[...]) + rpbd_ref[...]
        z_ref[...] = z
        zr_ref[...] = jnp.maximum(z, 0.0)


def _prop_node_body(agg_ref, pec_ref, ppb_ref, wp12_ref, out_ref):
    pe = jnp.maximum(pec_ref[...] + _mm(agg_ref[...], ppb_ref[...]), 0.0)
    out_ref[...] = _mm(pe, wp12_ref[...])


def _final_body(agg_ref, pec_ref, ppb_ref, fw0_ref, fb0_ref, fw1_ref,
                fb1_ref, fw2_ref, fb2_ref, out_ref):
    pe = jnp.maximum(pec_ref[...] + _mm(agg_ref[...], ppb_ref[...]), 0.0)
    p = jnp.maximum(_mm(pe, fw0_ref[...]) + fb0_ref[...], 0.0)
    p = jnp.maximum(_mm(p, fw1_ref[...]) + fb1_ref[...], 0.0)
    out_ref[...] = _mm(p, fw2_ref[...]) + fb2_ref[...]


def _make_gather_kernel(n, e):
    eq = e // 4
    nchunks = eq // CB
    mesh = plsc.VectorSubcoreMesh(core_axis_name="c", subcore_axis_name="s")

    @functools.partial(
        pl.kernel,
        out_type=jax.ShapeDtypeStruct((eq, 128), jnp.float32),
        mesh=mesh,
        scratch_types=[
            pltpu.VMEM((CB,), jnp.int32),
            pltpu.VMEM((CB, 16), jnp.float32),
            pltpu.SemaphoreType.DMA,
        ],
        compiler_params=_SC_PARAMS,
    )
    def gather_kernel(nt_hbm, recv_hbm, send_hbm, rs_hbm, idx_v, rows_v, sem):
        c = lax.axis_index("c")
        s = lax.axis_index("s")
        wid = s * NC + c

        def rounds(k, carry):
            ch = wid + k * (NC * NS)

            @pl.when(ch < nchunks)
            def _():
                base = ch * CB
                for q in range(4):
                    for ep, col in ((recv_hbm, q * 32), (send_hbm, q * 32 + 16)):
                        pltpu.sync_copy(ep.at[pl.ds(q * eq + base, CB)], idx_v)
                        pltpu.async_copy(nt_hbm.at[idx_v], rows_v, sem).wait()
                        pltpu.sync_copy(
                            rows_v, rs_hbm.at[pl.ds(base, CB), pl.ds(col, 16)])
            return carry

        nrounds = (nchunks + NC * NS - 1) // (NC * NS)
        lax.fori_loop(0, nrounds, rounds, 0)

    return gather_kernel


def _add_relu_chunk(buf, gbuf, rows):
    def body(i, carry):
        for r in range(2):
            for col in (0, 16):
                sl = (2 * i + r, pl.ds(col, 16))
                buf[sl] = jnp.maximum(buf[sl] + gbuf[sl], 0.0)
        return carry
    lax.fori_loop(0, rows // 2, body, 0)


def _zero_rows(buf, rows):
    def body(i, carry):
        buf[i, pl.ds(0, 16)] = jnp.zeros((16,), jnp.float32)
        buf[i, pl.ds(16, 16)] = jnp.zeros((16,), jnp.float32)
        return carry
    lax.fori_loop(0, rows, body, 0)


def _agg_writeout(acc, agg_hbm, zbuf, c, s, stripe, stripe_pieces):
    pltpu.sync_copy(acc.at[pl.ds(s * stripe, stripe)],
                    agg_hbm.at[pl.ds(s * stripe, stripe),
                               pl.ds(c * HALF, HALF)])
    _zero_rows(zbuf, min(stripe_pieces[0][1], stripe))
    for r0, rcnt in stripe_pieces:
        pltpu.sync_copy(
            zbuf.at[pl.ds(0, rcnt)],
            agg_hbm.at[pl.ds(s * stripe + r0, rcnt),
                       pl.ds(NF + c * HALF, HALF)])


def _offset_idx(dst_r, src_r, dst_s, src_s, cn):
    starts = sorted({min(j * 16, CD - 16) for j in range((CD + 15) // 16)})
    for st in starts:
        dst_r[pl.ds(st, 16)] = src_r[pl.ds(st, 16)] + cn
        dst_s[pl.ds(st, 16)] = src_s[pl.ds(st, 16)] + cn


def _pipelined_stream_loop(*, nchunks, row00, z_hbm, colbase, idx_ofs,
                           recv_hbm, send_hbm, p1f_hbm, p2f_hbm, acc, cn,
                           idxr_v, idxrg_v, idxs_v, idxsg_v, zbuf, gbuf, lsem, gsem,
                           pipelined=True):
    with_gather = p1f_hbm is not None

    if not pipelined:
        def chunk(k, carry):
            base = row00 + k * CD
            pltpu.sync_copy(recv_hbm.at[pl.ds(idx_ofs + base, CD)], idxr_v[0])
            pltpu.sync_copy(z_hbm.at[pl.ds(base, CD), pl.ds(colbase, HALF)],
                            zbuf[0])
            if with_gather:
                pltpu.sync_copy(send_hbm.at[pl.ds(idx_ofs + base, CD)],
                                idxs_v[0])

                _offset_idx(idxrg_v[0], idxr_v[0], idxsg_v[0], idxs_v[0],
                            cn)
                cp1 = pltpu.async_copy(p1f_hbm.at[idxrg_v[0]], zbuf[0],
                                       gsem[0], add=True)
                cp2 = pltpu.async_copy(p2f_hbm.at[idxsg_v[0]], gbuf[0],
                                       gsem[1])
                cp1.wait()
                cp2.wait()
                _add_relu_chunk(zbuf[0], gbuf[0], CD)
            pltpu.sync_copy(zbuf[0], acc.at[idxr_v[0]], add=True)
            return carry
        lax.fori_loop(0, nchunks, chunk, 0)
        return

    def start_loads(k, b):
        base = row00 + k * CD
        pltpu.async_copy(recv_hbm.at[pl.ds(idx_ofs + base, CD)], idxr_v[b],
                         lsem[b])
        pltpu.async_copy(z_hbm.at[pl.ds(base, CD), pl.ds(colbase, HALF)],
                         zbuf[b], lsem[b])
        if with_gather:
            pltpu.async_copy(send_hbm.at[pl.ds(idx_ofs + base, CD)],
                             idxs_v[b], lsem[b])

    def wait_loads(b):
        pltpu.make_async_copy(recv_hbm.at[pl.ds(idx_ofs, CD)], idxr_v[b],
                              lsem[b]).wait()
        pltpu.make_async_copy(z_hbm.at[pl.ds(0, CD), pl.ds(colbase, HALF)],
                              zbuf[b], lsem[b]).wait()
        if with_gather:
            pltpu.make_async_copy(send_hbm.at[pl.ds(idx_ofs, CD)], idxs_v[b],
                                  lsem[b]).wait()

    start_loads(0, 0)

    def k2body(k2, carry):
        for b in (0, 1):
            k = 2 * k2 + b
            wait_loads(b)
            if with_gather:
                _offset_idx(idxrg_v[b], idxr_v[b], idxsg_v[b], idxs_v[b], cn)
                cp1 = pltpu.async_copy(p1f_hbm.at[idxrg_v[b]], zbuf[b],
                                       gsem[b], add=True)
                cp2 = pltpu.async_copy(p2f_hbm.at[idxsg_v[b]], gbuf[b],
                                       gsem[b])

            @pl.when(k + 1 < nchunks)
            def _(k=k, b=b):
                start_loads(k + 1, 1 - b)

            if with_gather:
                cp1.wait()
                cp2.wait()
                _add_relu_chunk(zbuf[b], gbuf[b], CD)
            pltpu.sync_copy(zbuf[b], acc.at[idxr_v[b]], add=True)
        return carry

    lax.fori_loop(0, nchunks // 2, k2body, 0)


def _make_scatter1_kernel(n, e):
    eq = e // 4
    ept = eq // 4
    nchunks = ept // CD
    stripe = n // NS
    stripe_pieces = [(r0, min(CD, stripe - r0)) for r0 in range(0, stripe, CD)]
    mesh = plsc.VectorSubcoreMesh(core_axis_name="c", subcore_axis_name="s")

    @functools.partial(
        pl.kernel,
        out_type=jax.ShapeDtypeStruct((n, 128), jnp.float32),
        mesh=mesh,
        scratch_types=[
            pltpu.VMEM_SHARED((n, HALF), jnp.float32),
            [pltpu.VMEM((CD,), jnp.int32)] * 2,
            [pltpu.VMEM((CD, HALF), jnp.float32)] * 2,
            [pltpu.SemaphoreType.DMA] * 2,
        ],
        compiler_params=_SC_PARAMS,
    )
    def fn(za_hbm, zb_hbm, recv_hbm, agg_hbm, acc, idxr_v, zbuf, lsem):
        c = lax.axis_index("c")
        s = lax.axis_index("s")

        _zero_rows(zbuf[0], min(CD, stripe))
        for r0, rcnt in stripe_pieces:
            pltpu.sync_copy(zbuf[0].at[pl.ds(0, rcnt)],
                            acc.at[pl.ds(s * stripe + r0, rcnt)])
        plsc.subcore_barrier()

        sid = s // 4
        row00 = (s % 4) * ept
        for q, (zref, qhalf) in enumerate(((za_hbm, 0), (za_hbm, 1),
                                           (zb_hbm, 0), (zb_hbm, 1))):
            @pl.when(sid == q)
            def _(zref=zref, qhalf=qhalf, q=q):
                _pipelined_stream_loop(
                    nchunks=nchunks, row00=row00, z_hbm=zref,
                    colbase=qhalf * NF + c * HALF, idx_ofs=q * eq,
                    recv_hbm=recv_hbm, send_hbm=None, p1f_hbm=None,
                    p2f_hbm=None, acc=acc, cn=None, idxr_v=idxr_v,
                    idxrg_v=None, idxs_v=None, idxsg_v=None, zbuf=zbuf,
                    gbuf=None,
                    lsem=lsem, gsem=None)

        plsc.subcore_barrier()
        _agg_writeout(acc, agg_hbm, zbuf[0], c, s, stripe, stripe_pieces)

    return fn


def _make_scatter2_kernel(n, e):
    eq = e // 4
    ept = eq // 4
    nchunks = ept // CD
    stripe = n // NS
    stripe_pieces = [(r0, min(CD, stripe - r0)) for r0 in range(0, stripe, CD)]
    mesh = plsc.VectorSubcoreMesh(core_axis_name="c", subcore_axis_name="s")

    @functools.partial(
        pl.kernel,
        out_type=[jax.ShapeDtypeStruct((n, 128), jnp.float32),
                  jax.ShapeDtypeStruct((NC * n, HALF), jnp.float32),
                  jax.ShapeDtypeStruct((NC * n, HALF), jnp.float32)],
        mesh=mesh,
        scratch_types=[
            pltpu.VMEM_SHARED((n, HALF), jnp.float32),
            [pltpu.VMEM((CD,), jnp.int32)] * 2,
            [pltpu.VMEM((CD,), jnp.int32)] * 2,
            [pltpu.VMEM((CD,), jnp.int32)] * 2,
            [pltpu.VMEM((CD,), jnp.int32)] * 2,
            [pltpu.VMEM((CD, HALF), jnp.float32)] * 2,
            [pltpu.VMEM((CD, HALF), jnp.float32)] * 2,
            [pltpu.SemaphoreType.DMA] * 2,
            [pltpu.SemaphoreType.DMA] * 2,
        ],
        compiler_params=_SC_PARAMS,
    )
    def fn(za_hbm, zb_hbm, recv_hbm, send_hbm, p12_hbm, agg_hbm, p1f_hbm,
           p2f_hbm, acc, idxr_v, idxrg_v, idxs_v, idxsg_v, zbuf, gbuf, lsem,
           gsem):
        c = lax.axis_index("c")
        s = lax.axis_index("s")
        cn = (c * n).astype(jnp.int32)

        _zero_rows(zbuf[0], min(CD, stripe))
        for r0, rcnt in stripe_pieces:
            pltpu.sync_copy(zbuf[0].at[pl.ds(0, rcnt)],
                            acc.at[pl.ds(s * stripe + r0, rcnt)])

        for src_col, dst in ((c * HALF, p1f_hbm), (NF + c * HALF, p2f_hbm)):
            for r0, rcnt in stripe_pieces:
                pltpu.sync_copy(
                    p12_hbm.at[pl.ds(s * stripe + r0, rcnt),
                               pl.ds(src_col, HALF)],
                    zbuf[0].at[pl.ds(0, rcnt)])
                pltpu.sync_copy(zbuf[0].at[pl.ds(0, rcnt)],
                                dst.at[pl.ds(cn + s * stripe + r0, rcnt)])

        plsc.subcore_barrier()

        sid = s // 4
        row00 = (s % 4) * ept
        for q, (zref, qhalf) in enumerate(((za_hbm, 0), (za_hbm, 1),
                                           (zb_hbm, 0), (zb_hbm, 1))):
            @pl.when(sid == q)
            def _(zref=zref, qhalf=qhalf, q=q):
                _pipelined_stream_loop(
                    nchunks=nchunks, row00=row00, z_hbm=zref,
                    colbase=qhalf * NF + c * HALF, idx_ofs=q * eq,
                    recv_hbm=recv_hbm, send_hbm=send_hbm, p1f_hbm=p1f_hbm,
                    p2f_hbm=p2f_hbm, acc=acc, cn=cn, idxr_v=idxr_v,
                    idxrg_v=idxrg_v, idxs_v=idxs_v, idxsg_v=idxsg_v,
                    zbuf=zbuf, gbuf=gbuf,
                    lsem=lsem, gsem=gsem)

        plsc.subcore_barrier()
        _agg_writeout(acc, agg_hbm, zbuf[0], c, s, stripe, stripe_pieces)

    return fn


def kernel(state, attr, receivers, senders, Ra, pe_w0, pe_b0, pe_w1, pe_b1,
           re_w0, re_b0, re_w1, re_b1, re_w2, re_b2, rp_w, rp_b, pp_w, pp_b,
           fp_w0, fp_b0, fp_w1, fp_b1, fp_w2, fp_b2):
    n = state.shape[0]
    e = receivers.shape[0]
    eq = e // 4
    f32 = jnp.float32
    r1 = lambda b: b.reshape(1, -1)
    zz = lambda r, c: jnp.zeros((r, c), f32)

    nt = jnp.concatenate([attr, state, zz(n, 3)], axis=1)

    pe_w0p = jnp.concatenate([pe_w0[0:7], pe_w0[13:19], zz(3, NF)], axis=0)
    w0r = jnp.concatenate([re_w0[0:7], re_w0[26:32], zz(3, NF)], axis=0)
    w0s = jnp.concatenate([re_w0[13:20], re_w0[32:38], zz(3, NF)], axis=0)
    w_ra = re_w0[38:39]

    def dup(w):
        k = w.shape[0]
        return jnp.concatenate(
            [jnp.concatenate([w, zz(k, NF)], 1),
             jnp.concatenate([zz(k, NF), w], 1)], 0)

    wqa = jnp.concatenate([dup1 for dup1 in (
        jnp.concatenate([w0r, zz(16, NF)], 1),
        jnp.concatenate([w0s, zz(16, NF)], 1),
        jnp.concatenate([zz(16, NF), w0r], 1),
        jnp.concatenate([zz(16, NF), w0s], 1),
        zz(64, 2 * NF))], axis=0)
    wqb = jnp.concatenate([
        zz(64, 2 * NF),
        jnp.concatenate([w0r, zz(16, NF)], 1),
        jnp.concatenate([w0s, zz(16, NF)], 1),
        jnp.concatenate([zz(16, NF), w0r], 1),
        jnp.concatenate([zz(16, NF), w0s], 1)], axis=0)
    wra2 = dup(w_ra)
    w1d, w2d = dup(re_w1), dup(re_w2)
    wad = dup(rp_w[0:NF])
    b0d = r1(jnp.concatenate([re_b0, re_b0]))
    b1d = r1(jnp.concatenate([re_b1, re_b1]))
    b2d = r1(jnp.concatenate([re_b2, re_b2]))
    rpbd = r1(jnp.concatenate([rp_b, rp_b]))

    w_p12 = jnp.concatenate([rp_w[NF:2 * NF], rp_w[2 * NF:3 * NF]], axis=1)
    ppb128 = jnp.concatenate([pp_w[NF:2 * NF], zz(NF, NF)], axis=0)
    ra_a = jnp.concatenate([Ra[0:eq], Ra[eq:2 * eq]], axis=1)
    ra_b = jnp.concatenate([Ra[2 * eq:3 * eq], Ra[3 * eq:]], axis=1)

    nb = n // BN
    pec = pl.pallas_call(
        _node_encode_body,
        grid=(nb,),
        in_specs=[pl.BlockSpec((BN, 16), lambda i: (i, 0)),
                  _full_spec((16, NF)), _full_spec((1, NF)),
                  _full_spec((NF, NF)), _full_spec((1, NF)),
                  _full_spec((NF, NF)), _full_spec((1, NF))],
        out_specs=pl.BlockSpec((BN, NF), lambda i: (i, 0)),
        out_shape=jax.ShapeDtypeStruct((n, NF), f32),
    )(nt, pe_w0p, r1(pe_b0), pe_w1, r1(pe_b1), pp_w[0:NF], r1(pp_b))

    rs = _make_gather_kernel(n, e)(nt, receivers, senders)

    z_a, z_b, zr_a, zr_b = pl.pallas_call(
        _rel_encode_body,
        grid=(eq // BQ,),
        in_specs=[pl.BlockSpec((BQ, 128), lambda i: (i, 0)),
                  pl.BlockSpec((BQ, 2), lambda i: (i, 0)),
                  pl.BlockSpec((BQ, 2), lambda i: (i, 0)),
                  _full_spec((128, 128)), _full_spec((128, 128)),
                  _full_spec((2, 128)), _full_spec((1, 128)),
                  _full_spec((128, 128)), _full_spec((1, 128)),
                  _full_spec((128, 128)), _full_spec((1, 128)),
                  _full_spec((128, 128)), _full_spec((1, 128))],
        out_specs=[pl.BlockSpec((BQ, 128), lambda i: (i, 0)),
                   pl.BlockSpec((BQ, 128), lambda i: (i, 0)),
                   pl.BlockSpec((BQ, 128), lambda i: (i, 0)),
                   pl.BlockSpec((BQ, 128), lambda i: (i, 0))],
        out_shape=[jax.ShapeDtypeStruct((eq, 128), f32),
                   jax.ShapeDtypeStruct((eq, 128), f32),
                   jax.ShapeDtypeStruct((eq, 128), f32),
                   jax.ShapeDtypeStruct((eq, 128), f32)],
    )(rs, ra_a, ra_b, wqa, wqb, wra2, b0d, w1d, b1d, w2d, b2d, wad, rpbd)

    agg1 = _make_scatter1_kernel(n, e)(zr_a, zr_b, receivers)

    p12 = pl.pallas_call(
        _prop_node_body,
        grid=(nb,),
        in_specs=[pl.BlockSpec((BN, 128), lambda i: (i, 0)),
                  pl.BlockSpec((BN, NF), lambda i: (i, 0)),
                  _full_spec((128, NF)), _full_spec((NF, 128))],
        out_specs=pl.BlockSpec((BN, 128), lambda i: (i, 0)),
        out_shape=jax.ShapeDtypeStruct((n, 128), f32),
    )(agg1, pec, ppb128, w_p12)

    agg2, _, _ = _make_scatter2_kernel(n, e)(
        z_a, z_b, receivers, senders, p12)

    pred = pl.pallas_call(
        _final_body,
        grid=(nb,),
        in_specs=[pl.BlockSpec((BN, 128), lambda i: (i, 0)),
                  pl.BlockSpec((BN, NF), lambda i: (i, 0)),
                  _full_spec((128, NF)),
                  _full_spec((NF, NF)), _full_spec((1, NF)),
                  _full_spec((NF, NF)), _full_spec((1, NF)),
                  _full_spec((NF, 3)), _full_spec((1, 3))],
        out_specs=pl.BlockSpec((BN, 3), lambda i: (i, 0)),
        out_shape=jax.ShapeDtypeStruct((n, 3), f32),
    )(agg2, pec, ppb128, fp_w0, r1(fp_b0), fp_w1, r1(fp_b1),
      fp_w2, r1(fp_b2))

    return pred


# speedup vs baseline: 12.6191x; 1.0352x over previous
import functools

import jax
import jax.numpy as jnp
from jax import lax
from jax.experimental import pallas as pl
from jax.experimental.pallas import tpu as pltpu
from jax.experimental.pallas import tpu_sc as plsc

NF = 64
HALF = 32
NC = 2
NS = 16
BN = 2000
BQ = 2000
CB = 2000
CD = 200

_SC_PARAMS = pltpu.CompilerParams(use_tc_tiling_on_sc=False)


def _full_spec(shape):
    return pl.BlockSpec(shape, lambda i: tuple(0 for _ in shape))


def _mm(a, b):
    return jnp.dot(a, b, preferred_element_type=jnp.float32)


def _node_encode_body(nt_ref, w0_ref, b0_ref, w1_ref, b1_ref, wpa_ref,
                      bpa_ref, out_ref):
    h = jnp.maximum(_mm(nt_ref[...], w0_ref[...]) + b0_ref[...], 0.0)
    pe = jnp.maximum(_mm(h, w1_ref[...]) + b1_ref[...], 0.0)
    out_ref[...] = _mm(pe, wpa_ref[...]) + bpa_ref[...]


def _rel_encode_body(rs_ref, raa_ref, rab_ref, wqa_ref, wqb_ref, wra2_ref,
                     b0d_ref, w1d_ref, b1d_ref, w2d_ref, b2d_ref, wad_ref,
                     rpbd_ref, za_ref, zb_ref, zra_ref, zrb_ref):
    blk = rs_ref[...]
    wra2 = wra2_ref[...]
    for ra_ref, wq_ref, z_ref, zr_ref in ((raa_ref, wqa_ref, za_ref, zra_ref),
                                          (rab_ref, wqb_ref, zb_ref, zrb_ref)):
        h = _mm(blk, wq_ref[...]) + _mm(ra_ref[...], wra2) + b0d_ref[...]
        h = jnp.maximum(h, 0.0)
        h = jnp.maximum(_mm(h, w1d_ref[...]) + b1d_ref[...], 0.0)
        rel = jnp.maximum(_mm(h, w2d_ref[...]) + b2d_ref[...], 0.0)
        z = _mm(rel, wad_ref[...]) + rpbd_ref[...]
        z_ref[...] = z
        zr_ref[...] = jnp.maximum(z, 0.0)


def _prop_node_body(agg_ref, pec_ref, ppb_ref, wp12_ref, out_ref):
    pe = jnp.maximum(pec_ref[...] + _mm(agg_ref[...], ppb_ref[...]), 0.0)
    out_ref[...] = _mm(pe, wp12_ref[...])


def _final_body(agg_ref, pec_ref, ppb_ref, fw0_ref, fb0_ref, fw1_ref,
                fb1_ref, fw2_ref, fb2_ref, out_ref):
    pe = jnp.maximum(pec_ref[...] + _mm(agg_ref[...], ppb_ref[...]), 0.0)
    p = jnp.maximum(_mm(pe, fw0_ref[...]) + fb0_ref[...], 0.0)
    p = jnp.maximum(_mm(p, fw1_ref[...]) + fb1_ref[...], 0.0)
    out_ref[...] = _mm(p, fw2_ref[...]) + fb2_ref[...]


def _make_gather_kernel(n, e):
    eq = e // 4
    nchunks = eq // CB
    mesh = plsc.VectorSubcoreMesh(core_axis_name="c", subcore_axis_name="s")

    @functools.partial(
        pl.kernel,
        out_type=jax.ShapeDtypeStruct((eq, 128), jnp.float32),
        mesh=mesh,
        scratch_types=[
            [pltpu.VMEM((CB,), jnp.int32)] * 2,
            [pltpu.VMEM((CB, 16), jnp.float32)] * 2,
            [pltpu.SemaphoreType.DMA] * 2,
            [pltpu.SemaphoreType.DMA] * 2,
            [pltpu.SemaphoreType.DMA] * 2,
        ],
        compiler_params=_SC_PARAMS,
    )
    def gather_kernel(nt_hbm, recv_hbm, send_hbm, rs_hbm, idx_v, rows_v,
                      isem, gsem, wsem):
        c = lax.axis_index("c")
        s = lax.axis_index("s")
        wid = s * NC + c

        parts = [(q, ep_i, q * 32 + 16 * ep_i)
                 for q in range(4) for ep_i in range(2)]

        def rounds(k, carry):
            ch = wid + k * (NC * NS)

            @pl.when(ch < nchunks)
            def _():
                base = ch * CB

                def idx_src(p):
                    q, ep_i, _ = parts[p]
                    ep = (recv_hbm, send_hbm)[ep_i]
                    return ep.at[pl.ds(q * eq + base, CB)]

                pltpu.async_copy(idx_src(0), idx_v[0], isem[0])
                writes = [None, None]
                for p, (q, ep_i, col) in enumerate(parts):
                    b = p % 2
                    if writes[b] is not None:
                        writes[b].wait()
                    pltpu.make_async_copy(idx_src(p), idx_v[b],
                                          isem[b]).wait()
                    gcp = pltpu.async_copy(nt_hbm.at[idx_v[b]], rows_v[b],
                                           gsem[b])
                    if p + 1 < len(parts):
                        pltpu.async_copy(idx_src(p + 1), idx_v[1 - b],
                                         isem[1 - b])
                    gcp.wait()
                    writes[b] = pltpu.async_copy(
                        rows_v[b],
                        rs_hbm.at[pl.ds(base, CB), pl.ds(col, 16)], wsem[b])
                for w in writes:
                    w.wait()
            return carry

        nrounds = (nchunks + NC * NS - 1) // (NC * NS)
        lax.fori_loop(0, nrounds, rounds, 0)

    return gather_kernel


def _add_relu_chunk(buf, gbuf, rows):
    def body(i, carry):
        for r in range(2):
            for col in (0, 16):
                sl = (2 * i + r, pl.ds(col, 16))
                buf[sl] = jnp.maximum(buf[sl] + gbuf[sl], 0.0)
        return carry
    lax.fori_loop(0, rows // 2, body, 0)


def _zero_rows(buf, rows):
    def body(i, carry):
        buf[i, pl.ds(0, 16)] = jnp.zeros((16,), jnp.float32)
        buf[i, pl.ds(16, 16)] = jnp.zeros((16,), jnp.float32)
        return carry
    lax.fori_loop(0, rows, body, 0)


def _agg_writeout(acc, agg_hbm, zbuf, c, s, stripe, stripe_pieces):
    pltpu.sync_copy(acc.at[pl.ds(s * stripe, stripe)],
                    agg_hbm.at[pl.ds(s * stripe, stripe),
                               pl.ds(c * HALF, HALF)])
    _zero_rows(zbuf, min(stripe_pieces[0][1], stripe))
    for r0, rcnt in stripe_pieces:
        pltpu.sync_copy(
            zbuf.at[pl.ds(0, rcnt)],
            agg_hbm.at[pl.ds(s * stripe + r0, rcnt),
                       pl.ds(NF + c * HALF, HALF)])


def _offset_idx(dst_r, src_r, dst_s, src_s, cn):
    starts = sorted({min(j * 16, CD - 16) for j in range((CD + 15) // 16)})
    for st in starts:
        dst_r[pl.ds(st, 16)] = src_r[pl.ds(st, 16)] + cn
        dst_s[pl.ds(st, 16)] = src_s[pl.ds(st, 16)] + cn


def _pipelined_stream_loop(*, nchunks, row00, z_hbm, colbase, idx_ofs,
                           recv_hbm, send_hbm, p1f_hbm, p2f_hbm, acc, cn,
                           idxr_v, idxrg_v, idxs_v, idxsg_v, zbuf, gbuf, lsem, gsem,
                           pipelined=True):
    with_gather = p1f_hbm is not None

    if not pipelined:
        def chunk(k, carry):
            base = row00 + k * CD
            pltpu.sync_copy(recv_hbm.at[pl.ds(idx_ofs + base, CD)], idxr_v[0])
            pltpu.sync_copy(z_hbm.at[pl.ds(base, CD), pl.ds(colbase, HALF)],
                            zbuf[0])
            if with_gather:
                pltpu.sync_copy(send_hbm.at[pl.ds(idx_ofs + base, CD)],
                                idxs_v[0])

                _offset_idx(idxrg_v[0], idxr_v[0], idxsg_v[0], idxs_v[0],
                            cn)
                cp1 = pltpu.async_copy(p1f_hbm.at[idxrg_v[0]], zbuf[0],
                                       gsem[0], add=True)
                cp2 = pltpu.async_copy(p2f_hbm.at[idxsg_v[0]], gbuf[0],
                                       gsem[1])
                cp1.wait()
                cp2.wait()
                _add_relu_chunk(zbuf[0], gbuf[0], CD)
            pltpu.sync_copy(zbuf[0], acc.at[idxr_v[0]], add=True)
            return carry
        lax.fori_loop(0, nchunks, chunk, 0)
        return

    def start_loads(k, b):
        base = row00 + k * CD
        pltpu.async_copy(recv_hbm.at[pl.ds(idx_ofs + base, CD)], idxr_v[b],
                         lsem[b])
        pltpu.async_copy(z_hbm.at[pl.ds(base, CD), pl.ds(colbase, HALF)],
                         zbuf[b], lsem[b])
        if with_gather:
            pltpu.async_copy(send_hbm.at[pl.ds(idx_ofs + base, CD)],
                             idxs_v[b], lsem[b])

    def wait_loads(b):
        pltpu.make_async_copy(recv_hbm.at[pl.ds(idx_ofs, CD)], idxr_v[b],
                              lsem[b]).wait()
        pltpu.make_async_copy(z_hbm.at[pl.ds(0, CD), pl.ds(colbase, HALF)],
                              zbuf[b], lsem[b]).wait()
        if with_gather:
            pltpu.make_async_copy(send_hbm.at[pl.ds(idx_ofs, CD)], idxs_v[b],
                                  lsem[b]).wait()

    start_loads(0, 0)

    def k2body(k2, carry):
        for b in (0, 1):
            k = 2 * k2 + b
            wait_loads(b)
            if with_gather:
                _offset_idx(idxrg_v[b], idxr_v[b], idxsg_v[b], idxs_v[b], cn)
                cp1 = pltpu.async_copy(p1f_hbm.at[idxrg_v[b]], zbuf[b],
                                       gsem[b], add=True)
                cp2 = pltpu.async_copy(p2f_hbm.at[idxsg_v[b]], gbuf[b],
                                       gsem[b])

            @pl.when(k + 1 < nchunks)
            def _(k=k, b=b):
                start_loads(k + 1, 1 - b)

            if with_gather:
                cp1.wait()
                cp2.wait()
                _add_relu_chunk(zbuf[b], gbuf[b], CD)
            pltpu.sync_copy(zbuf[b], acc.at[idxr_v[b]], add=True)
        return carry

    lax.fori_loop(0, nchunks // 2, k2body, 0)


def _make_scatter1_kernel(n, e):
    eq = e // 4
    ept = eq // 4
    nchunks = ept // CD
    stripe = n // NS
    stripe_pieces = [(r0, min(CD, stripe - r0)) for r0 in range(0, stripe, CD)]
    mesh = plsc.VectorSubcoreMesh(core_axis_name="c", subcore_axis_name="s")

    @functools.partial(
        pl.kernel,
        out_type=jax.ShapeDtypeStruct((n, 128), jnp.float32),
        mesh=mesh,
        scratch_types=[
            pltpu.VMEM_SHARED((n, HALF), jnp.float32),
            [pltpu.VMEM((CD,), jnp.int32)] * 2,
            [pltpu.VMEM((CD, HALF), jnp.float32)] * 2,
            [pltpu.SemaphoreType.DMA] * 2,
        ],
        compiler_params=_SC_PARAMS,
    )
    def fn(za_hbm, zb_hbm, recv_hbm, agg_hbm, acc, idxr_v, zbuf, lsem):
        c = lax.axis_index("c")
        s = lax.axis_index("s")

        _zero_rows(zbuf[0], min(CD, stripe))
        for r0, rcnt in stripe_pieces:
            pltpu.sync_copy(zbuf[0].at[pl.ds(0, rcnt)],
                            acc.at[pl.ds(s * stripe + r0, rcnt)])
        plsc.subcore_barrier()

        sid = s // 4
        row00 = (s % 4) * ept
        for q, (zref, qhalf) in enumerate(((za_hbm, 0), (za_hbm, 1),
                                           (zb_hbm, 0), (zb_hbm, 1))):
            @pl.when(sid == q)
            def _(zref=zref, qhalf=qhalf, q=q):
                _pipelined_stream_loop(
                    nchunks=nchunks, row00=row00, z_hbm=zref,
                    colbase=qhalf * NF + c * HALF, idx_ofs=q * eq,
                    recv_hbm=recv_hbm, send_hbm=None, p1f_hbm=None,
                    p2f_hbm=None, acc=acc, cn=None, idxr_v=idxr_v,
                    idxrg_v=None, idxs_v=None, idxsg_v=None, zbuf=zbuf,
                    gbuf=None,
                    lsem=lsem, gsem=None)

        plsc.subcore_barrier()
        _agg_writeout(acc, agg_hbm, zbuf[0], c, s, stripe, stripe_pieces)

    return fn


def _make_scatter2_kernel(n, e):
    eq = e // 4
    ept = eq // 4
    nchunks = ept // CD
    stripe = n // NS
    stripe_pieces = [(r0, min(CD, stripe - r0)) for r0 in range(0, stripe, CD)]
    mesh = plsc.VectorSubcoreMesh(core_axis_name="c", subcore_axis_name="s")

    @functools.partial(
        pl.kernel,
        out_type=[jax.ShapeDtypeStruct((n, 128), jnp.float32),
                  jax.ShapeDtypeStruct((NC * n, HALF), jnp.float32),
                  jax.ShapeDtypeStruct((NC * n, HALF), jnp.float32)],
        mesh=mesh,
        scratch_types=[
            pltpu.VMEM_SHARED((n, HALF), jnp.float32),
            [pltpu.VMEM((CD,), jnp.int32)] * 2,
            [pltpu.VMEM((CD,), jnp.int32)] * 2,
            [pltpu.VMEM((CD,), jnp.int32)] * 2,
            [pltpu.VMEM((CD,), jnp.int32)] * 2,
            [pltpu.VMEM((CD, HALF), jnp.float32)] * 2,
            [pltpu.VMEM((CD, HALF), jnp.float32)] * 2,
            [pltpu.SemaphoreType.DMA] * 2,
            [pltpu.SemaphoreType.DMA] * 2,
        ],
        compiler_params=_SC_PARAMS,
    )
    def fn(za_hbm, zb_hbm, recv_hbm, send_hbm, p12_hbm, agg_hbm, p1f_hbm,
           p2f_hbm, acc, idxr_v, idxrg_v, idxs_v, idxsg_v, zbuf, gbuf, lsem,
           gsem):
        c = lax.axis_index("c")
        s = lax.axis_index("s")
        cn = (c * n).astype(jnp.int32)

        _zero_rows(zbuf[0], min(CD, stripe))
        for r0, rcnt in stripe_pieces:
            pltpu.sync_copy(zbuf[0].at[pl.ds(0, rcnt)],
                            acc.at[pl.ds(s * stripe + r0, rcnt)])

        for src_col, dst in ((c * HALF, p1f_hbm), (NF + c * HALF, p2f_hbm)):
            for r0, rcnt in stripe_pieces:
                pltpu.sync_copy(
                    p12_hbm.at[pl.ds(s * stripe + r0, rcnt),
                               pl.ds(src_col, HALF)],
                    zbuf[0].at[pl.ds(0, rcnt)])
                pltpu.sync_copy(zbuf[0].at[pl.ds(0, rcnt)],
                                dst.at[pl.ds(cn + s * stripe + r0, rcnt)])

        plsc.subcore_barrier()

        sid = s // 4
        row00 = (s % 4) * ept
        for q, (zref, qhalf) in enumerate(((za_hbm, 0), (za_hbm, 1),
                                           (zb_hbm, 0), (zb_hbm, 1))):
            @pl.when(sid == q)
            def _(zref=zref, qhalf=qhalf, q=q):
                _pipelined_stream_loop(
                    nchunks=nchunks, row00=row00, z_hbm=zref,
                    colbase=qhalf * NF + c * HALF, idx_ofs=q * eq,
                    recv_hbm=recv_hbm, send_hbm=send_hbm, p1f_hbm=p1f_hbm,
                    p2f_hbm=p2f_hbm, acc=acc, cn=cn, idxr_v=idxr_v,
                    idxrg_v=idxrg_v, idxs_v=idxs_v, idxsg_v=idxsg_v,
                    zbuf=zbuf, gbuf=gbuf,
                    lsem=lsem, gsem=gsem)

        plsc.subcore_barrier()
        _agg_writeout(acc, agg_hbm, zbuf[0], c, s, stripe, stripe_pieces)

    return fn


def kernel(state, attr, receivers, senders, Ra, pe_w0, pe_b0, pe_w1, pe_b1,
           re_w0, re_b0, re_w1, re_b1, re_w2, re_b2, rp_w, rp_b, pp_w, pp_b,
           fp_w0, fp_b0, fp_w1, fp_b1, fp_w2, fp_b2):
    n = state.shape[0]
    e = receivers.shape[0]
    eq = e // 4
    f32 = jnp.float32
    r1 = lambda b: b.reshape(1, -1)
    zz = lambda r, c: jnp.zeros((r, c), f32)

    nt = jnp.concatenate([attr, state, zz(n, 3)], axis=1)

    pe_w0p = jnp.concatenate([pe_w0[0:7], pe_w0[13:19], zz(3, NF)], axis=0)
    w0r = jnp.concatenate([re_w0[0:7], re_w0[26:32], zz(3, NF)], axis=0)
    w0s = jnp.concatenate([re_w0[13:20], re_w0[32:38], zz(3, NF)], axis=0)
    w_ra = re_w0[38:39]

    def dup(w):
        k = w.shape[0]
        return jnp.concatenate(
            [jnp.concatenate([w, zz(k, NF)], 1),
             jnp.concatenate([zz(k, NF), w], 1)], 0)

    wqa = jnp.concatenate([dup1 for dup1 in (
        jnp.concatenate([w0r, zz(16, NF)], 1),
        jnp.concatenate([w0s, zz(16, NF)], 1),
        jnp.concatenate([zz(16, NF), w0r], 1),
        jnp.concatenate([zz(16, NF), w0s], 1),
        zz(64, 2 * NF))], axis=0)
    wqb = jnp.concatenate([
        zz(64, 2 * NF),
        jnp.concatenate([w0r, zz(16, NF)], 1),
        jnp.concatenate([w0s, zz(16, NF)], 1),
        jnp.concatenate([zz(16, NF), w0r], 1),
        jnp.concatenate([zz(16, NF), w0s], 1)], axis=0)
    wra2 = dup(w_ra)
    w1d, w2d = dup(re_w1), dup(re_w2)
    wad = dup(rp_w[0:NF])
    b0d = r1(jnp.concatenate([re_b0, re_b0]))
    b1d = r1(jnp.concatenate([re_b1, re_b1]))
    b2d = r1(jnp.concatenate([re_b2, re_b2]))
    rpbd = r1(jnp.concatenate([rp_b, rp_b]))

    w_p12 = jnp.concatenate([rp_w[NF:2 * NF], rp_w[2 * NF:3 * NF]], axis=1)
    ppb128 = jnp.concatenate([pp_w[NF:2 * NF], zz(NF, NF)], axis=0)
    ra_a = jnp.concatenate([Ra[0:eq], Ra[eq:2 * eq]], axis=1)
    ra_b = jnp.concatenate([Ra[2 * eq:3 * eq], Ra[3 * eq:]], axis=1)

    nb = n // BN
    pec = pl.pallas_call(
        _node_encode_body,
        grid=(nb,),
        in_specs=[pl.BlockSpec((BN, 16), lambda i: (i, 0)),
                  _full_spec((16, NF)), _full_spec((1, NF)),
                  _full_spec((NF, NF)), _full_spec((1, NF)),
                  _full_spec((NF, NF)), _full_spec((1, NF))],
        out_specs=pl.BlockSpec((BN, NF), lambda i: (i, 0)),
        out_shape=jax.ShapeDtypeStruct((n, NF), f32),
    )(nt, pe_w0p, r1(pe_b0), pe_w1, r1(pe_b1), pp_w[0:NF], r1(pp_b))

    rs = _make_gather_kernel(n, e)(nt, receivers, senders)

    z_a, z_b, zr_a, zr_b = pl.pallas_call(
        _rel_encode_body,
        grid=(eq // BQ,),
        in_specs=[pl.BlockSpec((BQ, 128), lambda i: (i, 0)),
                  pl.BlockSpec((BQ, 2), lambda i: (i, 0)),
                  pl.BlockSpec((BQ, 2), lambda i: (i, 0)),
                  _full_spec((128, 128)), _full_spec((128, 128)),
                  _full_spec((2, 128)), _full_spec((1, 128)),
                  _full_spec((128, 128)), _full_spec((1, 128)),
                  _full_spec((128, 128)), _full_spec((1, 128)),
                  _full_spec((128, 128)), _full_spec((1, 128))],
        out_specs=[pl.BlockSpec((BQ, 128), lambda i: (i, 0)),
                   pl.BlockSpec((BQ, 128), lambda i: (i, 0)),
                   pl.BlockSpec((BQ, 128), lambda i: (i, 0)),
                   pl.BlockSpec((BQ, 128), lambda i: (i, 0))],
        out_shape=[jax.ShapeDtypeStruct((eq, 128), f32),
                   jax.ShapeDtypeStruct((eq, 128), f32),
                   jax.ShapeDtypeStruct((eq, 128), f32),
                   jax.ShapeDtypeStruct((eq, 128), f32)],
    )(rs, ra_a, ra_b, wqa, wqb, wra2, b0d, w1d, b1d, w2d, b2d, wad, rpbd)

    agg1 = _make_scatter1_kernel(n, e)(zr_a, zr_b, receivers)

    p12 = pl.pallas_call(
        _prop_node_body,
        grid=(nb,),
        in_specs=[pl.BlockSpec((BN, 128), lambda i: (i, 0)),
                  pl.BlockSpec((BN, NF), lambda i: (i, 0)),
                  _full_spec((128, NF)), _full_spec((NF, 128))],
        out_specs=pl.BlockSpec((BN, 128), lambda i: (i, 0)),
        out_shape=jax.ShapeDtypeStruct((n, 128), f32),
    )(agg1, pec, ppb128, w_p12)

    agg2, _, _ = _make_scatter2_kernel(n, e)(
        z_a, z_b, receivers, senders, p12)

    pred = pl.pallas_call(
        _final_body,
        grid=(nb,),
        in_specs=[pl.BlockSpec((BN, 128), lambda i: (i, 0)),
                  pl.BlockSpec((BN, NF), lambda i: (i, 0)),
                  _full_spec((128, NF)),
                  _full_spec((NF, NF)), _full_spec((1, NF)),
                  _full_spec((NF, NF)), _full_spec((1, NF)),
                  _full_spec((NF, 3)), _full_spec((1, 3))],
        out_specs=pl.BlockSpec((BN, 3), lambda i: (i, 0)),
        out_shape=jax.ShapeDtypeStruct((n, 3), f32),
    )(agg2, pec, ppb128, fp_w0, r1(fp_b0), fp_w1, r1(fp_b1),
      fp_w2, r1(fp_b2))

    return pred


# speedup vs baseline: 13.4244x; 1.0638x over previous
import functools

import jax
import jax.numpy as jnp
from jax import lax
from jax.experimental import pallas as pl
from jax.experimental.pallas import tpu as pltpu
from jax.experimental.pallas import tpu_sc as plsc

NF = 64
HALF = 32
NC = 2
NS = 16
BN = 2000
BQ = 2000
CB = 2000
CD = 200

_SC_PARAMS = pltpu.CompilerParams(use_tc_tiling_on_sc=False)


def _full_spec(shape):
    return pl.BlockSpec(shape, lambda i: tuple(0 for _ in shape))


def _mm(a, b):
    return jnp.dot(a, b, preferred_element_type=jnp.float32)


def _node_encode_body(nt_ref, w0_ref, b0_ref, w1_ref, b1_ref, wpa_ref,
                      bpa_ref, out_ref):
    h = jnp.maximum(_mm(nt_ref[...], w0_ref[...]) + b0_ref[...], 0.0)
    pe = jnp.maximum(_mm(h, w1_ref[...]) + b1_ref[...], 0.0)
    out_ref[...] = _mm(pe, wpa_ref[...]) + bpa_ref[...]


def _rel_encode_body(rs_ref, raa_ref, rab_ref, wqa_ref, wqb_ref, wra2_ref,
                     b0d_ref, w1d_ref, b1d_ref, w2d_ref, b2d_ref, wad_ref,
                     rpbd_ref, za_ref, zb_ref, zra_ref, zrb_ref):
    blk = rs_ref[...]
    wra2 = wra2_ref[...]
    for ra_ref, wq_ref, z_ref, zr_ref in ((raa_ref, wqa_ref, za_ref, zra_ref),
                                          (rab_ref, wqb_ref, zb_ref, zrb_ref)):
        h = _mm(blk, wq_ref[...]) + _mm(ra_ref[...], wra2) + b0d_ref[...]
        h = jnp.maximum(h, 0.0)
        h = jnp.maximum(_mm(h, w1d_ref[...]) + b1d_ref[...], 0.0)
        rel = jnp.maximum(_mm(h, w2d_ref[...]) + b2d_ref[...], 0.0)
        z = _mm(rel, wad_ref[...]) + rpbd_ref[...]
        z_ref[...] = z
        zr_ref[...] = jnp.maximum(z, 0.0)


def _prop_node_body(agg_ref, pec_ref, ppb_ref, wp12_ref, out_ref):
    pe = jnp.maximum(pec_ref[...] + _mm(agg_ref[...], ppb_ref[...]), 0.0)
    out_ref[...] = _mm(pe, wp12_ref[...])


def _final_body(agg_ref, pec_ref, ppb_ref, fw0_ref, fb0_ref, fw1_ref,
                fb1_ref, fw2_ref, fb2_ref, out_ref):
    pe = jnp.maximum(pec_ref[...] + _mm(agg_ref[...], ppb_ref[...]), 0.0)
    p = jnp.maximum(_mm(pe, fw0_ref[...]) + fb0_ref[...], 0.0)
    p = jnp.maximum(_mm(p, fw1_ref[...]) + fb1_ref[...], 0.0)
    out_ref[...] = _mm(p, fw2_ref[...]) + fb2_ref[...]


def _make_gather_kernel(n, e):
    eq = e // 4
    nchunks = eq // CB
    mesh = plsc.VectorSubcoreMesh(core_axis_name="c", subcore_axis_name="s")

    @functools.partial(
        pl.kernel,
        out_type=jax.ShapeDtypeStruct((eq, 128), jnp.float32),
        mesh=mesh,
        scratch_types=[
            [pltpu.VMEM((CB,), jnp.int32)] * 2,
            [pltpu.VMEM((CB, 16), jnp.float32)] * 2,
            [pltpu.SemaphoreType.DMA] * 2,
            [pltpu.SemaphoreType.DMA] * 2,
            [pltpu.SemaphoreType.DMA] * 2,
        ],
        compiler_params=_SC_PARAMS,
    )
    def gather_kernel(nt_hbm, recv_hbm, send_hbm, rs_hbm, idx_v, rows_v,
                      isem, gsem, wsem):
        c = lax.axis_index("c")
        s = lax.axis_index("s")
        wid = s * NC + c

        parts = [(q, ep_i, q * 32 + 16 * ep_i)
                 for q in range(4) for ep_i in range(2)]

        def rounds(k, carry):
            ch = wid + k * (NC * NS)

            @pl.when(ch < nchunks)
            def _():
                base = ch * CB

                def idx_src(p):
                    q, ep_i, _ = parts[p]
                    ep = (recv_hbm, send_hbm)[ep_i]
                    return ep.at[pl.ds(q * eq + base, CB)]

                pltpu.async_copy(idx_src(0), idx_v[0], isem[0])
                writes = [None, None]
                for p, (q, ep_i, col) in enumerate(parts):
                    b = p % 2
                    if writes[b] is not None:
                        writes[b].wait()
                    pltpu.make_async_copy(idx_src(p), idx_v[b],
                                          isem[b]).wait()
                    gcp = pltpu.async_copy(nt_hbm.at[idx_v[b]], rows_v[b],
                                           gsem[b])
                    if p + 1 < len(parts):
                        pltpu.async_copy(idx_src(p + 1), idx_v[1 - b],
                                         isem[1 - b])
                    gcp.wait()
                    writes[b] = pltpu.async_copy(
                        rows_v[b],
                        rs_hbm.at[pl.ds(base, CB), pl.ds(col, 16)], wsem[b])
                for w in writes:
                    w.wait()
            return carry

        nrounds = (nchunks + NC * NS - 1) // (NC * NS)
        lax.fori_loop(0, nrounds, rounds, 0)

    return gather_kernel


def _add_relu_chunk(buf, gbuf, rows):
    def body(i, carry):
        for r in range(2):
            for col in (0, 16):
                sl = (2 * i + r, pl.ds(col, 16))
                buf[sl] = jnp.maximum(buf[sl] + gbuf[sl], 0.0)
        return carry
    lax.fori_loop(0, rows // 2, body, 0)


def _zero_rows(buf, rows):
    def body(i, carry):
        buf[i, pl.ds(0, 16)] = jnp.zeros((16,), jnp.float32)
        buf[i, pl.ds(16, 16)] = jnp.zeros((16,), jnp.float32)
        return carry
    lax.fori_loop(0, rows, body, 0)


def _agg_writeout(acc, agg_hbm, zbuf, c, s, stripe, stripe_pieces):
    pltpu.sync_copy(acc.at[pl.ds(s * stripe, stripe)],
                    agg_hbm.at[pl.ds(s * stripe, stripe),
                               pl.ds(c * HALF, HALF)])
    _zero_rows(zbuf, min(stripe_pieces[0][1], stripe))
    for r0, rcnt in stripe_pieces:
        pltpu.sync_copy(
            zbuf.at[pl.ds(0, rcnt)],
            agg_hbm.at[pl.ds(s * stripe + r0, rcnt),
                       pl.ds(NF + c * HALF, HALF)])


def _offset_idx(dst_r, src_r, dst_s, src_s, cn):
    starts = sorted({min(j * 16, CD - 16) for j in range((CD + 15) // 16)})
    for st in starts:
        dst_r[pl.ds(st, 16)] = src_r[pl.ds(st, 16)] + cn
        dst_s[pl.ds(st, 16)] = src_s[pl.ds(st, 16)] + cn


def _pipelined_stream_loop(*, nchunks, row00, z_hbm, colbase, idx_ofs,
                           recv_hbm, send_hbm, p1f_hbm, p2f_hbm, acc, cn,
                           idxr_v, idxrg_v, idxs_v, idxsg_v, zbuf, gbuf, lsem, gsem,
                           ssem, pipelined=True):
    with_gather = p1f_hbm is not None

    if not pipelined:
        def chunk(k, carry):
            base = row00 + k * CD
            pltpu.sync_copy(recv_hbm.at[pl.ds(idx_ofs + base, CD)], idxr_v[0])
            pltpu.sync_copy(z_hbm.at[pl.ds(base, CD), pl.ds(colbase, HALF)],
                            zbuf[0])
            if with_gather:
                pltpu.sync_copy(send_hbm.at[pl.ds(idx_ofs + base, CD)],
                                idxs_v[0])

                _offset_idx(idxrg_v[0], idxr_v[0], idxsg_v[0], idxs_v[0],
                            cn)
                cp1 = pltpu.async_copy(p1f_hbm.at[idxrg_v[0]], zbuf[0],
                                       gsem[0], add=True)
                cp2 = pltpu.async_copy(p2f_hbm.at[idxsg_v[0]], gbuf[0],
                                       gsem[1])
                cp1.wait()
                cp2.wait()
                _add_relu_chunk(zbuf[0], gbuf[0], CD)
            pltpu.sync_copy(zbuf[0], acc.at[idxr_v[0]], add=True)
            return carry
        lax.fori_loop(0, nchunks, chunk, 0)
        return

    def start_loads(k, b):
        base = row00 + k * CD
        pltpu.async_copy(recv_hbm.at[pl.ds(idx_ofs + base, CD)], idxr_v[b],
                         lsem[b])
        pltpu.async_copy(z_hbm.at[pl.ds(base, CD), pl.ds(colbase, HALF)],
                         zbuf[b], lsem[b])
        if with_gather:
            pltpu.async_copy(send_hbm.at[pl.ds(idx_ofs + base, CD)],
                             idxs_v[b], lsem[b])

    def wait_loads(b):
        pltpu.make_async_copy(recv_hbm.at[pl.ds(idx_ofs, CD)], idxr_v[b],
                              lsem[b]).wait()
        pltpu.make_async_copy(z_hbm.at[pl.ds(0, CD), pl.ds(colbase, HALF)],
                              zbuf[b], lsem[b]).wait()
        if with_gather:
            pltpu.make_async_copy(send_hbm.at[pl.ds(idx_ofs, CD)], idxs_v[b],
                                  lsem[b]).wait()

    start_loads(0, 0)

    def k2body(k2, carry):
        for b in (0, 1):
            k = 2 * k2 + b
            wait_loads(b)
            if with_gather:
                _offset_idx(idxrg_v[b], idxr_v[b], idxsg_v[b], idxs_v[b], cn)
                cp1 = pltpu.async_copy(p1f_hbm.at[idxrg_v[b]], zbuf[b],
                                       gsem[b], add=True)
                cp2 = pltpu.async_copy(p2f_hbm.at[idxsg_v[b]], gbuf[b],
                                       gsem[b])

            @pl.when((k + 1 < nchunks) & (k >= 1))
            def _(b=b):
                pltpu.make_async_copy(zbuf[1 - b], acc.at[idxr_v[1 - b]],
                                      ssem[1 - b]).wait()

            @pl.when(k + 1 < nchunks)
            def _(k=k, b=b):
                start_loads(k + 1, 1 - b)

            if with_gather:
                cp1.wait()
                cp2.wait()
                _add_relu_chunk(zbuf[b], gbuf[b], CD)
            pltpu.async_copy(zbuf[b], acc.at[idxr_v[b]], ssem[b], add=True)
        return carry

    lax.fori_loop(0, nchunks // 2, k2body, 0)
    for b in (0, 1):
        pltpu.make_async_copy(zbuf[b], acc.at[idxr_v[b]], ssem[b]).wait()


def _make_scatter1_kernel(n, e):
    eq = e // 4
    ept = eq // 4
    nchunks = ept // CD
    stripe = n // NS
    stripe_pieces = [(r0, min(CD, stripe - r0)) for r0 in range(0, stripe, CD)]
    mesh = plsc.VectorSubcoreMesh(core_axis_name="c", subcore_axis_name="s")

    @functools.partial(
        pl.kernel,
        out_type=jax.ShapeDtypeStruct((n, 128), jnp.float32),
        mesh=mesh,
        scratch_types=[
            pltpu.VMEM_SHARED((n, HALF), jnp.float32),
            [pltpu.VMEM((CD,), jnp.int32)] * 2,
            [pltpu.VMEM((CD, HALF), jnp.float32)] * 2,
            [pltpu.SemaphoreType.DMA] * 2,
            [pltpu.SemaphoreType.DMA] * 2,
        ],
        compiler_params=_SC_PARAMS,
    )
    def fn(za_hbm, zb_hbm, recv_hbm, agg_hbm, acc, idxr_v, zbuf, lsem, ssem):
        c = lax.axis_index("c")
        s = lax.axis_index("s")

        _zero_rows(zbuf[0], min(CD, stripe))
        for r0, rcnt in stripe_pieces:
            pltpu.sync_copy(zbuf[0].at[pl.ds(0, rcnt)],
                            acc.at[pl.ds(s * stripe + r0, rcnt)])
        plsc.subcore_barrier()

        sid = s // 4
        row00 = (s % 4) * ept
        for q, (zref, qhalf) in enumerate(((za_hbm, 0), (za_hbm, 1),
                                           (zb_hbm, 0), (zb_hbm, 1))):
            @pl.when(sid == q)
            def _(zref=zref, qhalf=qhalf, q=q):
                _pipelined_stream_loop(
                    nchunks=nchunks, row00=row00, z_hbm=zref,
                    colbase=qhalf * NF + c * HALF, idx_ofs=q * eq,
                    recv_hbm=recv_hbm, send_hbm=None, p1f_hbm=None,
                    p2f_hbm=None, acc=acc, cn=None, idxr_v=idxr_v,
                    idxrg_v=None, idxs_v=None, idxsg_v=None, zbuf=zbuf,
                    gbuf=None,
                    lsem=lsem, gsem=None, ssem=ssem)

        plsc.subcore_barrier()
        _agg_writeout(acc, agg_hbm, zbuf[0], c, s, stripe, stripe_pieces)

    return fn


def _make_scatter2_kernel(n, e):
    eq = e // 4
    ept = eq // 4
    nchunks = ept // CD
    stripe = n // NS
    stripe_pieces = [(r0, min(CD, stripe - r0)) for r0 in range(0, stripe, CD)]
    mesh = plsc.VectorSubcoreMesh(core_axis_name="c", subcore_axis_name="s")

    @functools.partial(
        pl.kernel,
        out_type=[jax.ShapeDtypeStruct((n, 128), jnp.float32),
                  jax.ShapeDtypeStruct((NC * n, HALF), jnp.float32),
                  jax.ShapeDtypeStruct((NC * n, HALF), jnp.float32)],
        mesh=mesh,
        scratch_types=[
            pltpu.VMEM_SHARED((n, HALF), jnp.float32),
            [pltpu.VMEM((CD,), jnp.int32)] * 2,
            [pltpu.VMEM((CD,), jnp.int32)] * 2,
            [pltpu.VMEM((CD,), jnp.int32)] * 2,
            [pltpu.VMEM((CD,), jnp.int32)] * 2,
            [pltpu.VMEM((CD, HALF), jnp.float32)] * 2,
            [pltpu.VMEM((CD, HALF), jnp.float32)] * 2,
            [pltpu.SemaphoreType.DMA] * 2,
            [pltpu.SemaphoreType.DMA] * 2,
            [pltpu.SemaphoreType.DMA] * 2,
        ],
        compiler_params=_SC_PARAMS,
    )
    def fn(za_hbm, zb_hbm, recv_hbm, send_hbm, p12_hbm, agg_hbm, p1f_hbm,
           p2f_hbm, acc, idxr_v, idxrg_v, idxs_v, idxsg_v, zbuf, gbuf, lsem,
           gsem, ssem):
        c = lax.axis_index("c")
        s = lax.axis_index("s")
        cn = (c * n).astype(jnp.int32)

        _zero_rows(zbuf[0], min(CD, stripe))
        for r0, rcnt in stripe_pieces:
            pltpu.sync_copy(zbuf[0].at[pl.ds(0, rcnt)],
                            acc.at[pl.ds(s * stripe + r0, rcnt)])

        for src_col, dst in ((c * HALF, p1f_hbm), (NF + c * HALF, p2f_hbm)):
            for r0, rcnt in stripe_pieces:
                pltpu.sync_copy(
                    p12_hbm.at[pl.ds(s * stripe + r0, rcnt),
                               pl.ds(src_col, HALF)],
                    zbuf[0].at[pl.ds(0, rcnt)])
                pltpu.sync_copy(zbuf[0].at[pl.ds(0, rcnt)],
                                dst.at[pl.ds(cn + s * stripe + r0, rcnt)])

        plsc.subcore_barrier()

        sid = s // 4
        row00 = (s % 4) * ept
        for q, (zref, qhalf) in enumerate(((za_hbm, 0), (za_hbm, 1),
                                           (zb_hbm, 0), (zb_hbm, 1))):
            @pl.when(sid == q)
            def _(zref=zref, qhalf=qhalf, q=q):
                _pipelined_stream_loop(
                    nchunks=nchunks, row00=row00, z_hbm=zref,
                    colbase=qhalf * NF + c * HALF, idx_ofs=q * eq,
                    recv_hbm=recv_hbm, send_hbm=send_hbm, p1f_hbm=p1f_hbm,
                    p2f_hbm=p2f_hbm, acc=acc, cn=cn, idxr_v=idxr_v,
                    idxrg_v=idxrg_v, idxs_v=idxs_v, idxsg_v=idxsg_v,
                    zbuf=zbuf, gbuf=gbuf,
                    lsem=lsem, gsem=gsem, ssem=ssem)

        plsc.subcore_barrier()
        _agg_writeout(acc, agg_hbm, zbuf[0], c, s, stripe, stripe_pieces)

    return fn


def kernel(state, attr, receivers, senders, Ra, pe_w0, pe_b0, pe_w1, pe_b1,
           re_w0, re_b0, re_w1, re_b1, re_w2, re_b2, rp_w, rp_b, pp_w, pp_b,
           fp_w0, fp_b0, fp_w1, fp_b1, fp_w2, fp_b2):
    n = state.shape[0]
    e = receivers.shape[0]
    eq = e // 4
    f32 = jnp.float32
    r1 = lambda b: b.reshape(1, -1)
    zz = lambda r, c: jnp.zeros((r, c), f32)

    nt = jnp.concatenate([attr, state, zz(n, 3)], axis=1)

    pe_w0p = jnp.concatenate([pe_w0[0:7], pe_w0[13:19], zz(3, NF)], axis=0)
    w0r = jnp.concatenate([re_w0[0:7], re_w0[26:32], zz(3, NF)], axis=0)
    w0s = jnp.concatenate([re_w0[13:20], re_w0[32:38], zz(3, NF)], axis=0)
    w_ra = re_w0[38:39]

    def dup(w):
        k = w.shape[0]
        return jnp.concatenate(
            [jnp.concatenate([w, zz(k, NF)], 1),
             jnp.concatenate([zz(k, NF), w], 1)], 0)

    wqa = jnp.concatenate([dup1 for dup1 in (
        jnp.concatenate([w0r, zz(16, NF)], 1),
        jnp.concatenate([w0s, zz(16, NF)], 1),
        jnp.concatenate([zz(16, NF), w0r], 1),
        jnp.concatenate([zz(16, NF), w0s], 1),
        zz(64, 2 * NF))], axis=0)
    wqb = jnp.concatenate([
        zz(64, 2 * NF),
        jnp.concatenate([w0r, zz(16, NF)], 1),
        jnp.concatenate([w0s, zz(16, NF)], 1),
        jnp.concatenate([zz(16, NF), w0r], 1),
        jnp.concatenate([zz(16, NF), w0s], 1)], axis=0)
    wra2 = dup(w_ra)
    w1d, w2d = dup(re_w1), dup(re_w2)
    wad = dup(rp_w[0:NF])
    b0d = r1(jnp.concatenate([re_b0, re_b0]))
    b1d = r1(jnp.concatenate([re_b1, re_b1]))
    b2d = r1(jnp.concatenate([re_b2, re_b2]))
    rpbd = r1(jnp.concatenate([rp_b, rp_b]))

    w_p12 = jnp.concatenate([rp_w[NF:2 * NF], rp_w[2 * NF:3 * NF]], axis=1)
    ppb128 = jnp.concatenate([pp_w[NF:2 * NF], zz(NF, NF)], axis=0)
    ra_a = jnp.concatenate([Ra[0:eq], Ra[eq:2 * eq]], axis=1)
    ra_b = jnp.concatenate([Ra[2 * eq:3 * eq], Ra[3 * eq:]], axis=1)

    nb = n // BN
    pec = pl.pallas_call(
        _node_encode_body,
        grid=(nb,),
        in_specs=[pl.BlockSpec((BN, 16), lambda i: (i, 0)),
                  _full_spec((16, NF)), _full_spec((1, NF)),
                  _full_spec((NF, NF)), _full_spec((1, NF)),
                  _full_spec((NF, NF)), _full_spec((1, NF))],
        out_specs=pl.BlockSpec((BN, NF), lambda i: (i, 0)),
        out_shape=jax.ShapeDtypeStruct((n, NF), f32),
    )(nt, pe_w0p, r1(pe_b0), pe_w1, r1(pe_b1), pp_w[0:NF], r1(pp_b))

    rs = _make_gather_kernel(n, e)(nt, receivers, senders)

    z_a, z_b, zr_a, zr_b = pl.pallas_call(
        _rel_encode_body,
        grid=(eq // BQ,),
        in_specs=[pl.BlockSpec((BQ, 128), lambda i: (i, 0)),
                  pl.BlockSpec((BQ, 2), lambda i: (i, 0)),
                  pl.BlockSpec((BQ, 2), lambda i: (i, 0)),
                  _full_spec((128, 128)), _full_spec((128, 128)),
                  _full_spec((2, 128)), _full_spec((1, 128)),
                  _full_spec((128, 128)), _full_spec((1, 128)),
                  _full_spec((128, 128)), _full_spec((1, 128)),
                  _full_spec((128, 128)), _full_spec((1, 128))],
        out_specs=[pl.BlockSpec((BQ, 128), lambda i: (i, 0)),
                   pl.BlockSpec((BQ, 128), lambda i: (i, 0)),
                   pl.BlockSpec((BQ, 128), lambda i: (i, 0)),
                   pl.BlockSpec((BQ, 128), lambda i: (i, 0))],
        out_shape=[jax.ShapeDtypeStruct((eq, 128), f32),
                   jax.ShapeDtypeStruct((eq, 128), f32),
                   jax.ShapeDtypeStruct((eq, 128), f32),
                   jax.ShapeDtypeStruct((eq, 128), f32)],
    )(rs, ra_a, ra_b, wqa, wqb, wra2, b0d, w1d, b1d, w2d, b2d, wad, rpbd)

    agg1 = _make_scatter1_kernel(n, e)(zr_a, zr_b, receivers)

    p12 = pl.pallas_call(
        _prop_node_body,
        grid=(nb,),
        in_specs=[pl.BlockSpec((BN, 128), lambda i: (i, 0)),
                  pl.BlockSpec((BN, NF), lambda i: (i, 0)),
                  _full_spec((128, NF)), _full_spec((NF, 128))],
        out_specs=pl.BlockSpec((BN, 128), lambda i: (i, 0)),
        out_shape=jax.ShapeDtypeStruct((n, 128), f32),
    )(agg1, pec, ppb128, w_p12)

    agg2, _, _ = _make_scatter2_kernel(n, e)(
        z_a, z_b, receivers, senders, p12)

    pred = pl.pallas_call(
        _final_body,
        grid=(nb,),
        in_specs=[pl.BlockSpec((BN, 128), lambda i: (i, 0)),
                  pl.BlockSpec((BN, NF), lambda i: (i, 0)),
                  _full_spec((128, NF)),
                  _full_spec((NF, NF)), _full_spec((1, NF)),
                  _full_spec((NF, NF)), _full_spec((1, NF)),
                  _full_spec((NF, 3)), _full_spec((1, 3))],
        out_specs=pl.BlockSpec((BN, 3), lambda i: (i, 0)),
        out_shape=jax.ShapeDtypeStruct((n, 3), f32),
    )(agg2, pec, ppb128, fp_w0, r1(fp_b0), fp_w1, r1(fp_b1),
      fp_w2, r1(fp_b2))

    return pred


# speedup vs baseline: 13.4409x; 1.0012x over previous
import functools

import jax
import jax.numpy as jnp
from jax import lax
from jax.experimental import pallas as pl
from jax.experimental.pallas import tpu as pltpu
from jax.experimental.pallas import tpu_sc as plsc

NF = 64
HALF = 32
NC = 2
NS = 16
BN = 2000
BQ = 2000
CB = 2000
CD = 200

_SC_PARAMS = pltpu.CompilerParams(use_tc_tiling_on_sc=False)


def _full_spec(shape):
    return pl.BlockSpec(shape, lambda i: tuple(0 for _ in shape))


def _mm(a, b):
    return jnp.dot(a, b, preferred_element_type=jnp.float32)


def _node_encode_body(nt_ref, w0_ref, b0_ref, w1_ref, b1_ref, wpa_ref,
                      bpa_ref, out_ref):
    h = jnp.maximum(_mm(nt_ref[...], w0_ref[...]) + b0_ref[...], 0.0)
    pe = jnp.maximum(_mm(h, w1_ref[...]) + b1_ref[...], 0.0)
    out_ref[...] = _mm(pe, wpa_ref[...]) + bpa_ref[...]


def _rel_encode_body(rs_ref, raa_ref, rab_ref, wqa_ref, wqb_ref, wra2_ref,
                     b0d_ref, w1d_ref, b1d_ref, w2d_ref, b2d_ref, wad_ref,
                     rpbd_ref, za_ref, zb_ref, zra_ref, zrb_ref):
    blk = rs_ref[...]
    wra2 = wra2_ref[...]
    for ra_ref, wq_ref, z_ref, zr_ref in ((raa_ref, wqa_ref, za_ref, zra_ref),
                                          (rab_ref, wqb_ref, zb_ref, zrb_ref)):
        h = _mm(blk, wq_ref[...]) + _mm(ra_ref[...], wra2) + b0d_ref[...]
        h = jnp.maximum(h, 0.0)
        h = jnp.maximum(_mm(h, w1d_ref[...]) + b1d_ref[...], 0.0)
        rel = jnp.maximum(_mm(h, w2d_ref[...]) + b2d_ref[...], 0.0)
        z = _mm(rel, wad_ref[...]) + rpbd_ref[...]
        z_ref[...] = z
        zr_ref[...] = jnp.maximum(z, 0.0)


def _prop_node_body(agg_ref, pec_ref, ppb_ref, wp12_ref, out_ref):
    pe = jnp.maximum(pec_ref[...] + _mm(agg_ref[...], ppb_ref[...]), 0.0)
    out_ref[...] = _mm(pe, wp12_ref[...])


def _final_body(agg_ref, pec_ref, ppb_ref, fw0_ref, fb0_ref, fw1_ref,
                fb1_ref, fw2_ref, fb2_ref, out_ref):
    pe = jnp.maximum(pec_ref[...] + _mm(agg_ref[...], ppb_ref[...]), 0.0)
    p = jnp.maximum(_mm(pe, fw0_ref[...]) + fb0_ref[...], 0.0)
    p = jnp.maximum(_mm(p, fw1_ref[...]) + fb1_ref[...], 0.0)
    out_ref[...] = _mm(p, fw2_ref[...]) + fb2_ref[...]


def _make_gather_kernel(n, e):
    eq = e // 4
    nchunks = eq // CB
    mesh = plsc.VectorSubcoreMesh(core_axis_name="c", subcore_axis_name="s")

    @functools.partial(
        pl.kernel,
        out_type=jax.ShapeDtypeStruct((eq, 128), jnp.float32),
        mesh=mesh,
        scratch_types=[
            [pltpu.VMEM((CB,), jnp.int32)] * 2,
            [pltpu.VMEM((CB, 16), jnp.float32)] * 2,
            [pltpu.SemaphoreType.DMA] * 2,
            [pltpu.SemaphoreType.DMA] * 2,
            [pltpu.SemaphoreType.DMA] * 2,
        ],
        compiler_params=_SC_PARAMS,
    )
    def gather_kernel(nt_hbm, recv_hbm, send_hbm, rs_hbm, idx_v, rows_v,
                      isem, gsem, wsem):
        c = lax.axis_index("c")
        s = lax.axis_index("s")
        wid = s * NC + c

        parts = [(q, ep_i, q * 32 + 16 * ep_i)
                 for q in range(4) for ep_i in range(2)]

        def rounds(k, carry):
            ch = wid + k * (NC * NS)

            @pl.when(ch < nchunks)
            def _():
                base = ch * CB

                def idx_src(p):
                    q, ep_i, _ = parts[p]
                    ep = (recv_hbm, send_hbm)[ep_i]
                    return ep.at[pl.ds(q * eq + base, CB)]

                pltpu.async_copy(idx_src(0), idx_v[0], isem[0])
                writes = [None, None]
                for p, (q, ep_i, col) in enumerate(parts):
                    b = p % 2
                    if writes[b] is not None:
                        writes[b].wait()
                    pltpu.make_async_copy(idx_src(p), idx_v[b],
                                          isem[b]).wait()
                    gcp = pltpu.async_copy(nt_hbm.at[idx_v[b]], rows_v[b],
                                           gsem[b])
                    if p + 1 < len(parts):
                        pltpu.async_copy(idx_src(p + 1), idx_v[1 - b],
                                         isem[1 - b])
                    gcp.wait()
                    writes[b] = pltpu.async_copy(
                        rows_v[b],
                        rs_hbm.at[pl.ds(base, CB), pl.ds(col, 16)], wsem[b])
                for w in writes:
                    w.wait()
            return carry

        nrounds = (nchunks + NC * NS - 1) // (NC * NS)
        lax.fori_loop(0, nrounds, rounds, 0)

    return gather_kernel


def _add_relu_chunk(buf, gbuf, rows):
    def body(i, carry):
        for r in range(2):
            for col in (0, 16):
                sl = (2 * i + r, pl.ds(col, 16))
                buf[sl] = jnp.maximum(buf[sl] + gbuf[sl], 0.0)
        return carry
    lax.fori_loop(0, rows // 2, body, 0)


def _zero_rows(buf, rows):
    def body(i, carry):
        buf[i, pl.ds(0, 16)] = jnp.zeros((16,), jnp.float32)
        buf[i, pl.ds(16, 16)] = jnp.zeros((16,), jnp.float32)
        return carry
    lax.fori_loop(0, rows, body, 0)


def _agg_writeout(acc, agg_hbm, zbuf, c, s, stripe, stripe_pieces):
    pltpu.sync_copy(acc.at[pl.ds(s * stripe, stripe)],
                    agg_hbm.at[pl.ds(s * stripe, stripe),
                               pl.ds(c * HALF, HALF)])
    _zero_rows(zbuf, min(stripe_pieces[0][1], stripe))
    for r0, rcnt in stripe_pieces:
        pltpu.sync_copy(
            zbuf.at[pl.ds(0, rcnt)],
            agg_hbm.at[pl.ds(s * stripe + r0, rcnt),
                       pl.ds(NF + c * HALF, HALF)])


def _offset_idx(dst_r, src_r, dst_s, src_s, cn):
    starts = sorted({min(j * 16, CD - 16) for j in range((CD + 15) // 16)})
    for st in starts:
        dst_r[pl.ds(st, 16)] = src_r[pl.ds(st, 16)] + cn
        dst_s[pl.ds(st, 16)] = src_s[pl.ds(st, 16)] + cn


def _pipelined_stream_loop(*, nchunks, row00, z_hbm, colbase, idx_ofs,
                           recv_hbm, send_hbm, p1f_hbm, p2f_hbm, acc, cn,
                           idxr_v, idxrg_v, idxs_v, idxsg_v, zbuf, gbuf, lsem, gsem,
                           ssem):
    with_gather = p1f_hbm is not None

    def start_loads(k, b):
        base = row00 + k * CD
        pltpu.async_copy(recv_hbm.at[pl.ds(idx_ofs + base, CD)], idxr_v[b],
                         lsem[b])
        pltpu.async_copy(z_hbm.at[pl.ds(base, CD), pl.ds(colbase, HALF)],
                         zbuf[b], lsem[b])
        if with_gather:
            pltpu.async_copy(send_hbm.at[pl.ds(idx_ofs + base, CD)],
                             idxs_v[b], lsem[b])

    def wait_loads(b):
        pltpu.make_async_copy(recv_hbm.at[pl.ds(idx_ofs, CD)], idxr_v[b],
                              lsem[b]).wait()
        pltpu.make_async_copy(z_hbm.at[pl.ds(0, CD), pl.ds(colbase, HALF)],
                              zbuf[b], lsem[b]).wait()
        if with_gather:
            pltpu.make_async_copy(send_hbm.at[pl.ds(idx_ofs, CD)], idxs_v[b],
                                  lsem[b]).wait()

    start_loads(0, 0)

    def k2body(k2, carry):
        for b in (0, 1):
            k = 2 * k2 + b
            wait_loads(b)
            if with_gather:
                _offset_idx(idxrg_v[b], idxr_v[b], idxsg_v[b], idxs_v[b], cn)
                cp1 = pltpu.async_copy(p1f_hbm.at[idxrg_v[b]], zbuf[b],
                                       gsem[b], add=True)
                cp2 = pltpu.async_copy(p2f_hbm.at[idxsg_v[b]], gbuf[b],
                                       gsem[b])

            @pl.when((k + 1 < nchunks) & (k >= 1))
            def _(b=b):
                pltpu.make_async_copy(zbuf[1 - b], acc.at[idxr_v[1 - b]],
                                      ssem[1 - b]).wait()

            @pl.when(k + 1 < nchunks)
            def _(k=k, b=b):
                start_loads(k + 1, 1 - b)

            if with_gather:
                cp1.wait()
                cp2.wait()
                _add_relu_chunk(zbuf[b], gbuf[b], CD)
            pltpu.async_copy(zbuf[b], acc.at[idxr_v[b]], ssem[b], add=True)
        return carry

    lax.fori_loop(0, nchunks // 2, k2body, 0)
    for b in (0, 1):
        pltpu.make_async_copy(zbuf[b], acc.at[idxr_v[b]], ssem[b]).wait()


def _make_scatter1_kernel(n, e):
    eq = e // 4
    ept = eq // 4
    nchunks = ept // CD
    stripe = n // NS
    stripe_pieces = [(r0, min(CD, stripe - r0)) for r0 in range(0, stripe, CD)]
    mesh = plsc.VectorSubcoreMesh(core_axis_name="c", subcore_axis_name="s")

    @functools.partial(
        pl.kernel,
        out_type=jax.ShapeDtypeStruct((n, 128), jnp.float32),
        mesh=mesh,
        scratch_types=[
            pltpu.VMEM_SHARED((n, HALF), jnp.float32),
            [pltpu.VMEM((CD,), jnp.int32)] * 2,
            [pltpu.VMEM((CD, HALF), jnp.float32)] * 2,
            [pltpu.SemaphoreType.DMA] * 2,
            [pltpu.SemaphoreType.DMA] * 2,
        ],
        compiler_params=_SC_PARAMS,
    )
    def fn(za_hbm, zb_hbm, recv_hbm, agg_hbm, acc, idxr_v, zbuf, lsem, ssem):
        c = lax.axis_index("c")
        s = lax.axis_index("s")

        _zero_rows(zbuf[0], min(CD, stripe))
        for r0, rcnt in stripe_pieces:
            pltpu.sync_copy(zbuf[0].at[pl.ds(0, rcnt)],
                            acc.at[pl.ds(s * stripe + r0, rcnt)])
        plsc.subcore_barrier()

        sid = s // 4
        row00 = (s % 4) * ept
        for q, (zref, qhalf) in enumerate(((za_hbm, 0), (za_hbm, 1),
                                           (zb_hbm, 0), (zb_hbm, 1))):
            @pl.when(sid == q)
            def _(zref=zref, qhalf=qhalf, q=q):
                _pipelined_stream_loop(
                    nchunks=nchunks, row00=row00, z_hbm=zref,
                    colbase=qhalf * NF + c * HALF, idx_ofs=q * eq,
                    recv_hbm=recv_hbm, send_hbm=None, p1f_hbm=None,
                    p2f_hbm=None, acc=acc, cn=None, idxr_v=idxr_v,
                    idxrg_v=None, idxs_v=None, idxsg_v=None, zbuf=zbuf,
                    gbuf=None,
                    lsem=lsem, gsem=None, ssem=ssem)

        plsc.subcore_barrier()
        _agg_writeout(acc, agg_hbm, zbuf[0], c, s, stripe, stripe_pieces)

    return fn


def _make_scatter2_kernel(n, e):
    eq = e // 4
    ept = eq // 4
    nchunks = ept // CD
    stripe = n // NS
    stripe_pieces = [(r0, min(CD, stripe - r0)) for r0 in range(0, stripe, CD)]
    mesh = plsc.VectorSubcoreMesh(core_axis_name="c", subcore_axis_name="s")

    @functools.partial(
        pl.kernel,
        out_type=[jax.ShapeDtypeStruct((n, 128), jnp.float32),
                  jax.ShapeDtypeStruct((NC * n, HALF), jnp.float32),
                  jax.ShapeDtypeStruct((NC * n, HALF), jnp.float32)],
        mesh=mesh,
        scratch_types=[
            pltpu.VMEM_SHARED((n, HALF), jnp.float32),
            [pltpu.VMEM((CD,), jnp.int32)] * 2,
            [pltpu.VMEM((CD,), jnp.int32)] * 2,
            [pltpu.VMEM((CD,), jnp.int32)] * 2,
            [pltpu.VMEM((CD,), jnp.int32)] * 2,
            [pltpu.VMEM((CD, HALF), jnp.float32)] * 2,
            [pltpu.VMEM((CD, HALF), jnp.float32)] * 2,
            [pltpu.SemaphoreType.DMA] * 2,
            [pltpu.SemaphoreType.DMA] * 2,
            [pltpu.SemaphoreType.DMA] * 2,
        ],
        compiler_params=_SC_PARAMS,
    )
    def fn(za_hbm, zb_hbm, recv_hbm, send_hbm, p12_hbm, agg_hbm, p1f_hbm,
           p2f_hbm, acc, idxr_v, idxrg_v, idxs_v, idxsg_v, zbuf, gbuf, lsem,
           gsem, ssem):
        c = lax.axis_index("c")
        s = lax.axis_index("s")
        cn = (c * n).astype(jnp.int32)

        _zero_rows(zbuf[0], min(CD, stripe))
        for r0, rcnt in stripe_pieces:
            pltpu.sync_copy(zbuf[0].at[pl.ds(0, rcnt)],
                            acc.at[pl.ds(s * stripe + r0, rcnt)])

        for src_col, dst in ((c * HALF, p1f_hbm), (NF + c * HALF, p2f_hbm)):
            for r0, rcnt in stripe_pieces:
                pltpu.sync_copy(
                    p12_hbm.at[pl.ds(s * stripe + r0, rcnt),
                               pl.ds(src_col, HALF)],
                    zbuf[0].at[pl.ds(0, rcnt)])
                pltpu.sync_copy(zbuf[0].at[pl.ds(0, rcnt)],
                                dst.at[pl.ds(cn + s * stripe + r0, rcnt)])

        plsc.subcore_barrier()

        sid = s // 4
        row00 = (s % 4) * ept
        for q, (zref, qhalf) in enumerate(((za_hbm, 0), (za_hbm, 1),
                                           (zb_hbm, 0), (zb_hbm, 1))):
            @pl.when(sid == q)
            def _(zref=zref, qhalf=qhalf, q=q):
                _pipelined_stream_loop(
                    nchunks=nchunks, row00=row00, z_hbm=zref,
                    colbase=qhalf * NF + c * HALF, idx_ofs=q * eq,
                    recv_hbm=recv_hbm, send_hbm=send_hbm, p1f_hbm=p1f_hbm,
                    p2f_hbm=p2f_hbm, acc=acc, cn=cn, idxr_v=idxr_v,
                    idxrg_v=idxrg_v, idxs_v=idxs_v, idxsg_v=idxsg_v,
                    zbuf=zbuf, gbuf=gbuf,
                    lsem=lsem, gsem=gsem, ssem=ssem)

        plsc.subcore_barrier()
        _agg_writeout(acc, agg_hbm, zbuf[0], c, s, stripe, stripe_pieces)

    return fn


def kernel(state, attr, receivers, senders, Ra, pe_w0, pe_b0, pe_w1, pe_b1,
           re_w0, re_b0, re_w1, re_b1, re_w2, re_b2, rp_w, rp_b, pp_w, pp_b,
           fp_w0, fp_b0, fp_w1, fp_b1, fp_w2, fp_b2):
    n = state.shape[0]
    e = receivers.shape[0]
    eq = e // 4
    f32 = jnp.float32
    r1 = lambda b: b.reshape(1, -1)
    zz = lambda r, c: jnp.zeros((r, c), f32)

    nt = jnp.concatenate([attr, state, zz(n, 3)], axis=1)

    pe_w0p = jnp.concatenate([pe_w0[0:7], pe_w0[13:19], zz(3, NF)], axis=0)
    w0r = jnp.concatenate([re_w0[0:7], re_w0[26:32], zz(3, NF)], axis=0)
    w0s = jnp.concatenate([re_w0[13:20], re_w0[32:38], zz(3, NF)], axis=0)
    w_ra = re_w0[38:39]

    def dup(w):
        k = w.shape[0]
        return jnp.concatenate(
            [jnp.concatenate([w, zz(k, NF)], 1),
             jnp.concatenate([zz(k, NF), w], 1)], 0)

    wqa = jnp.concatenate([dup1 for dup1 in (
        jnp.concatenate([w0r, zz(16, NF)], 1),
        jnp.concatenate([w0s, zz(16, NF)], 1),
        jnp.concatenate([zz(16, NF), w0r], 1),
        jnp.concatenate([zz(16, NF), w0s], 1),
        zz(64, 2 * NF))], axis=0)
    wqb = jnp.concatenate([
        zz(64, 2 * NF),
        jnp.concatenate([w0r, zz(16, NF)], 1),
        jnp.concatenate([w0s, zz(16, NF)], 1),
        jnp.concatenate([zz(16, NF), w0r], 1),
        jnp.concatenate([zz(16, NF), w0s], 1)], axis=0)
    wra2 = dup(w_ra)
    w1d, w2d = dup(re_w1), dup(re_w2)
    wad = dup(rp_w[0:NF])
    b0d = r1(jnp.concatenate([re_b0, re_b0]))
    b1d = r1(jnp.concatenate([re_b1, re_b1]))
    b2d = r1(jnp.concatenate([re_b2, re_b2]))
    rpbd = r1(jnp.concatenate([rp_b, rp_b]))

    w_p12 = jnp.concatenate([rp_w[NF:2 * NF], rp_w[2 * NF:3 * NF]], axis=1)
    ppb128 = jnp.concatenate([pp_w[NF:2 * NF], zz(NF, NF)], axis=0)
    ra_a = jnp.concatenate([Ra[0:eq], Ra[eq:2 * eq]], axis=1)
    ra_b = jnp.concatenate([Ra[2 * eq:3 * eq], Ra[3 * eq:]], axis=1)

    nb = n // BN
    pec = pl.pallas_call(
        _node_encode_body,
        grid=(nb,),
        in_specs=[pl.BlockSpec((BN, 16), lambda i: (i, 0)),
                  _full_spec((16, NF)), _full_spec((1, NF)),
                  _full_spec((NF, NF)), _full_spec((1, NF)),
                  _full_spec((NF, NF)), _full_spec((1, NF))],
        out_specs=pl.BlockSpec((BN, NF), lambda i: (i, 0)),
        out_shape=jax.ShapeDtypeStruct((n, NF), f32),
    )(nt, pe_w0p, r1(pe_b0), pe_w1, r1(pe_b1), pp_w[0:NF], r1(pp_b))

    rs = _make_gather_kernel(n, e)(nt, receivers, senders)

    z_a, z_b, zr_a, zr_b = pl.pallas_call(
        _rel_encode_body,
        grid=(eq // BQ,),
        in_specs=[pl.BlockSpec((BQ, 128), lambda i: (i, 0)),
                  pl.BlockSpec((BQ, 2), lambda i: (i, 0)),
                  pl.BlockSpec((BQ, 2), lambda i: (i, 0)),
                  _full_spec((128, 128)), _full_spec((128, 128)),
                  _full_spec((2, 128)), _full_spec((1, 128)),
                  _full_spec((128, 128)), _full_spec((1, 128)),
                  _full_spec((128, 128)), _full_spec((1, 128)),
                  _full_spec((128, 128)), _full_spec((1, 128))],
        out_specs=[pl.BlockSpec((BQ, 128), lambda i: (i, 0)),
                   pl.BlockSpec((BQ, 128), lambda i: (i, 0)),
                   pl.BlockSpec((BQ, 128), lambda i: (i, 0)),
                   pl.BlockSpec((BQ, 128), lambda i: (i, 0))],
        out_shape=[jax.ShapeDtypeStruct((eq, 128), f32),
                   jax.ShapeDtypeStruct((eq, 128), f32),
                   jax.ShapeDtypeStruct((eq, 128), f32),
                   jax.ShapeDtypeStruct((eq, 128), f32)],
    )(rs, ra_a, ra_b, wqa, wqb, wra2, b0d, w1d, b1d, w2d, b2d, wad, rpbd)

    agg1 = _make_scatter1_kernel(n, e)(zr_a, zr_b, receivers)

    p12 = pl.pallas_call(
        _prop_node_body,
        grid=(nb,),
        in_specs=[pl.BlockSpec((BN, 128), lambda i: (i, 0)),
                  pl.BlockSpec((BN, NF), lambda i: (i, 0)),
                  _full_spec((128, NF)), _full_spec((NF, 128))],
        out_specs=pl.BlockSpec((BN, 128), lambda i: (i, 0)),
        out_shape=jax.ShapeDtypeStruct((n, 128), f32),
    )(agg1, pec, ppb128, w_p12)

    agg2, _, _ = _make_scatter2_kernel(n, e)(
        z_a, z_b, receivers, senders, p12)

    pred = pl.pallas_call(
        _final_body,
        grid=(nb,),
        in_specs=[pl.BlockSpec((BN, 128), lambda i: (i, 0)),
                  pl.BlockSpec((BN, NF), lambda i: (i, 0)),
                  _full_spec((128, NF)),
                  _full_spec((NF, NF)), _full_spec((1, NF)),
                  _full_spec((NF, NF)), _full_spec((1, NF)),
                  _full_spec((NF, 3)), _full_spec((1, 3))],
        out_specs=pl.BlockSpec((BN, 3), lambda i: (i, 0)),
        out_shape=jax.ShapeDtypeStruct((n, 3), f32),
    )(agg2, pec, ppb128, fp_w0, r1(fp_b0), fp_w1, r1(fp_b1),
      fp_w2, r1(fp_b2))

    return pred
